# baseline (speedup 1.0000x reference)
import functools

import jax
import jax.numpy as jnp
from jax import lax
from jax.experimental import pallas as pl
from jax.experimental.pallas import tpu as pltpu

N_USERS = 50000
N_ITEMS = 50000
N_WARM = 40000
EMB = 64
FEAT = 256
N_LAYERS = 2
N_NEGS = 16
B = 4096
NNZ = 1200000
N_NODES = N_USERS + N_WARM
TAU = 0.2
LAMBDA_CL = 0.5
ALIGN_W = 0.1
REG_W = 1e-4
EPS = 1e-12


_FP_BLK = 2000


def _feat_proj_body(feat_ref, w_ref, out_ref):
    x = feat_ref[...]
    w = w_ref[...]
    proj = lax.dot_general(x, w, (((1,), (1,)), ((), ())),
                           preferred_element_type=jnp.float32)
    nrm = jnp.sqrt(jnp.sum(proj * proj, axis=1, keepdims=True))
    out_ref[...] = proj / (nrm + EPS)


def _feat_proj(feat_warm, W):
    grid = N_WARM // _FP_BLK
    return pl.pallas_call(
        _feat_proj_body,
        grid=(grid,),
        in_specs=[
            pl.BlockSpec((_FP_BLK, FEAT), lambda i: (i, 0)),
            pl.BlockSpec((EMB, FEAT), lambda i: (0, 0)),
        ],
        out_specs=pl.BlockSpec((_FP_BLK, EMB), lambda i: (i, 0)),
        out_shape=jax.ShapeDtypeStruct((N_WARM, EMB), jnp.float32),
    )(feat_warm, W)


_L_BLK = 512


def _log_sigmoid(x):
    return jnp.minimum(x, 0.0) - jnp.log1p(jnp.exp(-jnp.abs(x)))


def _loss_body(u_ref, pos_ref, negt_ref, pf_ref, nft_ref, out_ref):
    i = pl.program_id(0)
    u = u_ref[...]
    pos = pos_ref[...]
    pf = pf_ref[...]

    pos_scores = jnp.sum(u * pos, axis=1)
    unrm = jnp.sqrt(jnp.sum(u * u, axis=1, keepdims=True))
    u_n = u / (unrm + EPS)
    pos_sim = jnp.sum(u_n * pf, axis=1) / TAU

    bpr_sum = jnp.float32(0.0)
    reg_neg = jnp.float32(0.0)
    m = pos_sim
    nf_sims = []
    for j in range(N_NEGS):
        ne = negt_ref[j]
        ns = jnp.sum(u * ne, axis=1)
        bpr_sum += jnp.sum(_log_sigmoid(pos_scores - ns))
        reg_neg += jnp.sum(ne * ne)
        nf = nft_ref[j]
        nfs = jnp.sum(u_n * nf, axis=1) / TAU
        nf_sims.append(nfs)
        m = jnp.maximum(m, nfs)
    s = jnp.exp(pos_sim - m)
    for j in range(N_NEGS):
        s += jnp.exp(nf_sims[j] - m)
    cl_sum = jnp.sum(jnp.log(s) + m - pos_sim)

    pnrm = jnp.sqrt(jnp.sum(pos * pos, axis=1, keepdims=True))
    pos_n = pos / (pnrm + EPS)
    diff = pos_n - pf
    align_sum = jnp.sum(diff * diff)

    reg_sum = jnp.sum(u * u) + jnp.sum(pos * pos) + reg_neg

    contrib = (-bpr_sum / (B * N_NEGS)
               + (LAMBDA_CL / B) * cl_sum
               + (ALIGN_W / (B * EMB)) * align_sum
               + (REG_W / B) * reg_sum)

    contrib2d = jnp.full((1, 1), 0.0, jnp.float32) + contrib

    @pl.when(i == 0)
    def _():
        out_ref[...] = contrib2d

    @pl.when(i > 0)
    def _():
        out_ref[...] = out_ref[...] + contrib2d


def _loss(u_e, pos_e, neg_t, pos_feat, neg_feat_t):
    grid = B // _L_BLK
    out = pl.pallas_call(
        _loss_body,
        grid=(grid,),
        in_specs=[
            pl.BlockSpec((_L_BLK, EMB), lambda i: (i, 0)),
            pl.BlockSpec((_L_BLK, EMB), lambda i: (i, 0)),
            pl.BlockSpec((N_NEGS, _L_BLK, EMB), lambda i: (0, i, 0)),
            pl.BlockSpec((_L_BLK, EMB), lambda i: (i, 0)),
            pl.BlockSpec((N_NEGS, _L_BLK, EMB), lambda i: (0, i, 0)),
        ],
        out_specs=pl.BlockSpec((1, 1), lambda i: (0, 0)),
        out_shape=jax.ShapeDtypeStruct((1, 1), jnp.float32),
    )(u_e, pos_e, neg_t, pos_feat, neg_feat_t)
    return out[0, 0]


def kernel(users, pos_items, neg_items, feat_all, user_emb, item_emb, W,
           adj_row, adj_col, adj_val, neg_feat_idx):
    all_emb = jnp.concatenate([user_emb, item_emb], axis=0)

    e = all_emb
    acc = all_emb
    for _ in range(N_LAYERS):
        msgs = adj_val[:, None] * jnp.take(e, adj_col, axis=0)
        e = jax.ops.segment_sum(msgs, adj_row, num_segments=N_NODES)
        acc = acc + e
    final_emb = acc / (N_LAYERS + 1)

    u_e = jnp.take(final_emb, users, axis=0)
    pos_e = jnp.take(final_emb, pos_items + N_USERS, axis=0)
    neg_t = jnp.take(final_emb, neg_items.T + N_USERS, axis=0)

    feat_proj = _feat_proj(feat_all[:N_WARM], W)
    pos_feat = jnp.take(feat_proj, pos_items, axis=0)
    neg_feat_t = jnp.take(feat_proj, neg_feat_idx.T, axis=0)

    return _loss(u_e, pos_e, neg_t, pos_feat, neg_feat_t)


# speedup vs baseline: 3.8552x; 3.8552x over previous
import functools

import jax
import jax.numpy as jnp
from jax import lax
from jax.experimental import pallas as pl
from jax.experimental.pallas import tpu as pltpu
from jax.experimental.pallas import tpu_sc as plsc

N_USERS = 50000
N_ITEMS = 50000
N_WARM = 40000
EMB = 64
FEAT = 256
N_LAYERS = 2
N_NEGS = 16
B = 4096
NNZ = 1200000
N_NODES = N_USERS + N_WARM
TAU = 0.2
LAMBDA_CL = 0.5
ALIGN_W = 0.1
REG_W = 1e-4
EPS = 1e-12

NW = 32
E_CH = 37504
K = 256
S = 1024
PAD_LEN = NW * E_CH + K
ND = EMB // 16

NA = B * (2 + N_NEGS)
NB = B * (1 + N_NEGS)
CA = NA // NW
CB = NB // NW
G = 128


def _zero_window(staging):
    def zb(i, _):
        for u in range(4):
            staging[pl.ds(i * 64 + u * 16, 16)] = jnp.zeros((16,), jnp.float32)
        return 0
    lax.fori_loop(0, S * EMB // 64, zb, 0)


def _spmm_body(row_hbm, col_hbm, val_hbm, src_hbm, out_hbm,
               cols_v, colsq_v, rows_v, vals_v, gbuf, staging, scal16, sem):
    wid = lax.axis_index("s") * 2 + lax.axis_index("c")
    start = wid * E_CH

    off0 = pl.multiple_of(jnp.maximum(start - 16, 0), 16)
    pltpu.sync_copy(row_hbm.at[pl.ds(off0, 16)], scal16)
    prev = jnp.where(wid == 0, jnp.int32(-1), scal16[...][15])
    off1 = pl.multiple_of(start + E_CH - 16, 16)
    pltpu.sync_copy(row_hbm.at[pl.ds(off1, 16)], scal16)
    last = jnp.minimum(scal16[...][15], jnp.int32(N_NODES - 1))

    _zero_window(staging)

    n_steps = ((PAD_LEN - start) // K + 1) + (last - prev + S - 1) // S + 2

    def step(_, state):
        pos, wbase, done = state

        def active(pos, wbase):
            wend = jnp.minimum(wbase + S, last + 1)
            pos_a = pl.multiple_of(pos - lax.rem(pos, 8), 8)
            pltpu.sync_copy(row_hbm.at[pl.ds(pos_a, K)], rows_v)
            pltpu.sync_copy(val_hbm.at[pl.ds(pos_a, K)], vals_v)
            pltpu.sync_copy(col_hbm.at[pl.ds(pos_a, K)], cols_v)

            def shift_body(g, _):
                colsq_v[pl.ds(g * 16, 16)] = lax.shift_right_logical(
                    cols_v[pl.ds(g * 16, 16)], 1)
                return 0

            lax.fori_loop(0, K // 16, shift_body, 0)
            cps = [pltpu.async_copy(src_hbm.at[colsq_v.at[pl.ds(j * 128, 128)]],
                                    gbuf.at[pl.ds(j * 128, 128), :], sem)
                   for j in range(K // 128)]
            for cp in cps:
                cp.wait()

            def group_body(g, cnt):
                rvec = rows_v[pl.ds(g * 16, 16)]
                vvec = vals_v[pl.ds(g * 16, 16)]
                hvec = jnp.bitwise_and(cols_v[pl.ds(g * 16, 16)], jnp.int32(1))
                m = (rvec >= wbase) & (rvec < wend)
                vmask = jnp.where(m, vvec, jnp.float32(0.0))
                offv = jnp.clip(rvec - wbase, 0, S - 1)
                lt_i = jnp.where(rvec < wend, jnp.int32(1), jnp.int32(0))
                gcnt = jnp.int32(0)
                for j in range(16):
                    gcnt = gcnt + lt_i[j]
                    off = offv[j]
                    vv = jnp.full((16,), vmask[j], jnp.float32)
                    e = g * 16 + j
                    base = pl.multiple_of(off * EMB, 16)
                    gb = pl.multiple_of(hvec[j] * EMB, 16)
                    for d in range(ND):
                        plsc.addupdate(staging.at[pl.ds(base + d * 16, 16)],
                                       vv * gbuf[e, pl.ds(gb + d * 16, 16)])
                return cnt + gcnt

            consumed = lax.fori_loop(0, K // 16, group_body, jnp.int32(0))
            batch_max = rows_v[pl.ds(K - 16, 16)][15]
            new_pos = pos_a + consumed
            adv = batch_max >= wend

            @pl.when(adv & (wbase + S <= last + 1))
            def _():
                dst = pl.multiple_of(wbase * EMB, 64)
                pltpu.sync_copy(staging, out_hbm.at[pl.ds(dst, S * EMB)])
                _zero_window(staging)

            @pl.when(adv & (wbase + S > last + 1))
            def _():
                n = last + 1 - wbase
                offd = jnp.int32(0)
                for bit in reversed(range(10)):
                    sz = 1 << bit
                    take = (n & sz) != 0

                    @pl.when(take)
                    def _(offd=offd, sz=sz):
                        so = pl.multiple_of(offd * EMB, 64)
                        do = pl.multiple_of((wbase + offd) * EMB, 64)
                        pltpu.sync_copy(staging.at[pl.ds(so, sz * EMB)],
                                        out_hbm.at[pl.ds(do, sz * EMB)])

                    offd = offd + jnp.where(take, jnp.int32(sz), jnp.int32(0))

            new_wbase = jnp.where(adv, wbase + S, wbase)
            new_done = jnp.where(new_wbase > last, jnp.int32(1), jnp.int32(0))
            return new_pos, new_wbase, new_done

        return lax.cond(done == 0, active, lambda p, w: (p, w, done), pos, wbase)

    lax.fori_loop(0, n_steps, step,
                  (start, prev + 1, jnp.where(prev >= last, 1, 0).astype(jnp.int32)))


def _spmm(adj_row_pad, adj_col_pad, adj_val_pad, src_2d):
    mesh = plsc.VectorSubcoreMesh(core_axis_name="c", subcore_axis_name="s")
    kfn = functools.partial(
        pl.kernel,
        out_type=jax.ShapeDtypeStruct((N_NODES * EMB,), jnp.float32),
        mesh=mesh,
        scratch_types=[
            pltpu.VMEM((K,), jnp.int32),
            pltpu.VMEM((K,), jnp.int32),
            pltpu.VMEM((K,), jnp.int32),
            pltpu.VMEM((K,), jnp.float32),
            pltpu.VMEM((K, 128), jnp.float32),
            pltpu.VMEM((S * EMB,), jnp.float32),
            pltpu.VMEM((16,), jnp.int32),
            pltpu.SemaphoreType.DMA,
        ],
    )(_spmm_body)
    return kfn(adj_row_pad, adj_col_pad, adj_val_pad, src_2d)


def _gather_body(qa_hbm, qb_hbm, e0_hbm, e1_hbm, e2_hbm, fp_hbm,
                 outa_hbm, outb_hbm, idxb, g0, g1, g2, acc, sem):
    wid = lax.axis_index("s") * 2 + lax.axis_index("c")
    base_a = wid * CA
    base_b = wid * CB

    for t in range(CA // G):
        off = pl.multiple_of(base_a + t * G, 128)
        pltpu.sync_copy(qa_hbm.at[pl.ds(off, G)], idxb)
        c0 = pltpu.async_copy(e0_hbm.at[idxb], g0, sem)
        c1 = pltpu.async_copy(e1_hbm.at[idxb], g1, sem)
        c2 = pltpu.async_copy(e2_hbm.at[idxb], g2, sem)
        c0.wait(); c1.wait(); c2.wait()

        def avg_row(r, _):
            for u in range(8):
                sl = pl.ds(u * 16, 16)
                acc[r, sl] = (g0[r, sl] + g1[r, sl] + g2[r, sl]) * (1.0 / 3.0)
            return 0

        lax.fori_loop(0, G, avg_row, 0)
        pltpu.sync_copy(acc, outa_hbm.at[pl.ds(off, G), :])

    for t in range(CB // G):
        off = pl.multiple_of(base_b + t * G, 128)
        pltpu.sync_copy(qb_hbm.at[pl.ds(off, G)], idxb)
        pltpu.async_copy(fp_hbm.at[idxb], g0, sem).wait()
        pltpu.sync_copy(g0, outb_hbm.at[pl.ds(off, G), :])


def _gather_batch(qa, qb, e0_2d, e1_2d, e2_2d, fp_2d):
    mesh = plsc.VectorSubcoreMesh(core_axis_name="c", subcore_axis_name="s")
    kfn = functools.partial(
        pl.kernel,
        out_type=(jax.ShapeDtypeStruct((NA, 128), jnp.float32),
                  jax.ShapeDtypeStruct((NB, 128), jnp.float32)),
        mesh=mesh,
        scratch_types=[
            pltpu.VMEM((G,), jnp.int32),
            pltpu.VMEM((G, 128), jnp.float32),
            pltpu.VMEM((G, 128), jnp.float32),
            pltpu.VMEM((G, 128), jnp.float32),
            pltpu.VMEM((G, 128), jnp.float32),
            pltpu.SemaphoreType.DMA,
        ],
    )(_gather_body)
    return kfn(qa, qb, e0_2d, e1_2d, e2_2d, fp_2d)


_FP_BLK = 2000


def _feat_proj_body(feat_ref, w_ref, out_ref):
    x = feat_ref[...]
    w = w_ref[...]
    proj = lax.dot_general(x, w, (((1,), (1,)), ((), ())),
                           preferred_element_type=jnp.float32)
    nrm = jnp.sqrt(jnp.sum(proj * proj, axis=1, keepdims=True))
    out_ref[...] = proj / (nrm + EPS)


def _feat_proj(feat_warm, W):
    grid = N_WARM // _FP_BLK
    return pl.pallas_call(
        _feat_proj_body,
        grid=(grid,),
        in_specs=[
            pl.BlockSpec((_FP_BLK, FEAT), lambda i: (i, 0)),
            pl.BlockSpec((EMB, FEAT), lambda i: (0, 0)),
        ],
        out_specs=pl.BlockSpec((_FP_BLK, EMB), lambda i: (i, 0)),
        out_shape=jax.ShapeDtypeStruct((N_WARM, EMB), jnp.float32),
    )(feat_warm, W)


_L_BLK = 512


def _log_sigmoid(x):
    return jnp.minimum(x, 0.0) - jnp.log1p(jnp.exp(-jnp.abs(x)))


def _sel(x128, hf):
    return jnp.where(hf < 0.5, x128[:, :EMB], x128[:, EMB:])


def _loss_body(u_ref, pos_ref, negt_ref, pf_ref, nft_ref,
               hu_ref, hp_ref, hn_ref, hnf_ref, out_ref):
    i = pl.program_id(0)
    u = _sel(u_ref[...], hu_ref[...])
    pos = _sel(pos_ref[...], hp_ref[...])
    pf = _sel(pf_ref[...], hp_ref[...])

    pos_scores = jnp.sum(u * pos, axis=1)
    unrm = jnp.sqrt(jnp.sum(u * u, axis=1, keepdims=True))
    u_n = u / (unrm + EPS)
    pos_sim = jnp.sum(u_n * pf, axis=1) / TAU

    bpr_sum = jnp.float32(0.0)
    reg_neg = jnp.float32(0.0)
    m = pos_sim
    nf_sims = []
    for j in range(N_NEGS):
        ne = _sel(negt_ref[j], hn_ref[j])
        ns = jnp.sum(u * ne, axis=1)
        bpr_sum += jnp.sum(_log_sigmoid(pos_scores - ns))
        reg_neg += jnp.sum(ne * ne)
        nf = _sel(nft_ref[j], hnf_ref[j])
        nfs = jnp.sum(u_n * nf, axis=1) / TAU
        nf_sims.append(nfs)
        m = jnp.maximum(m, nfs)
    s = jnp.exp(pos_sim - m)
    for j in range(N_NEGS):
        s += jnp.exp(nf_sims[j] - m)
    cl_sum = jnp.sum(jnp.log(s) + m - pos_sim)

    pnrm = jnp.sqrt(jnp.sum(pos * pos, axis=1, keepdims=True))
    pos_n = pos / (pnrm + EPS)
    diff = pos_n - pf
    align_sum = jnp.sum(diff * diff)

    reg_sum = jnp.sum(u * u) + jnp.sum(pos * pos) + reg_neg

    contrib = (-bpr_sum / (B * N_NEGS)
               + (LAMBDA_CL / B) * cl_sum
               + (ALIGN_W / (B * EMB)) * align_sum
               + (REG_W / B) * reg_sum)
    contrib2d = jnp.full((1, 1), 0.0, jnp.float32) + contrib

    @pl.when(i == 0)
    def _():
        out_ref[...] = contrib2d

    @pl.when(i > 0)
    def _():
        out_ref[...] = out_ref[...] + contrib2d


def _loss(u_p, pos_p, neg_p, pf_p, nf_p, hu, hp, hn, hnf):
    grid = B // _L_BLK
    out = pl.pallas_call(
        _loss_body,
        grid=(grid,),
        in_specs=[
            pl.BlockSpec((_L_BLK, 128), lambda i: (i, 0)),
            pl.BlockSpec((_L_BLK, 128), lambda i: (i, 0)),
            pl.BlockSpec((N_NEGS, _L_BLK, 128), lambda i: (0, i, 0)),
            pl.BlockSpec((_L_BLK, 128), lambda i: (i, 0)),
            pl.BlockSpec((N_NEGS, _L_BLK, 128), lambda i: (0, i, 0)),
            pl.BlockSpec((_L_BLK, 1), lambda i: (i, 0)),
            pl.BlockSpec((_L_BLK, 1), lambda i: (i, 0)),
            pl.BlockSpec((N_NEGS, _L_BLK, 1), lambda i: (0, i, 0)),
            pl.BlockSpec((N_NEGS, _L_BLK, 1), lambda i: (0, i, 0)),
        ],
        out_specs=pl.BlockSpec((1, 1), lambda i: (0, 0)),
        out_shape=jax.ShapeDtypeStruct((1, 1), jnp.float32),
    )(u_p, pos_p, neg_p, pf_p, nf_p, hu, hp, hn, hnf)
    return out[0, 0]


def kernel(users, pos_items, neg_items, feat_all, user_emb, item_emb, W,
           adj_row, adj_col, adj_val, neg_feat_idx):
    all_emb = jnp.concatenate([user_emb, item_emb], axis=0)
    e0_2d = all_emb.reshape(N_NODES * EMB // 128, 128)

    pad = PAD_LEN - NNZ
    row_p = jnp.concatenate(
        [adj_row.astype(jnp.int32), jnp.full((pad,), N_NODES, jnp.int32)])
    col_p = jnp.concatenate(
        [adj_col.astype(jnp.int32),
         (jnp.arange(pad, dtype=jnp.int32) % N_NODES)])
    val_p = jnp.concatenate([adj_val, jnp.zeros((pad,), jnp.float32)])

    e1_2d = _spmm(row_p, col_p, val_p, e0_2d).reshape(N_NODES * EMB // 128, 128)
    e2_2d = _spmm(row_p, col_p, val_p, e1_2d).reshape(N_NODES * EMB // 128, 128)

    users32 = users.astype(jnp.int32)
    pos32 = pos_items.astype(jnp.int32)
    neg32 = neg_items.astype(jnp.int32)
    negf32 = neg_feat_idx.astype(jnp.int32)

    idx_a = jnp.concatenate(
        [users32, pos32 + N_USERS, (neg32.T + N_USERS).reshape(-1)])
    idx_b = jnp.concatenate([pos32, negf32.T.reshape(-1)])
    qa, ha = idx_a >> 1, (idx_a & 1).astype(jnp.float32)
    qb, hb = idx_b >> 1, (idx_b & 1).astype(jnp.float32)

    fp = _feat_proj(feat_all[:N_WARM], W)
    fp_2d = fp.reshape(N_WARM * EMB // 128, 128)

    outa, outb = _gather_batch(qa, qb, e0_2d, e1_2d, e2_2d, fp_2d)

    u_p = outa[:B]
    pos_p = outa[B:2 * B]
    neg_p = outa[2 * B:].reshape(N_NEGS, B, 128)
    pf_p = outb[:B]
    nf_p = outb[B:].reshape(N_NEGS, B, 128)

    hu = ha[:B].reshape(B, 1)
    hp = ha[B:2 * B].reshape(B, 1)
    hn = ha[2 * B:].reshape(N_NEGS, B, 1)
    hnf = hb[B:].reshape(N_NEGS, B, 1)

    return _loss(u_p, pos_p, neg_p, pf_p, nf_p, hu, hp, hn, hnf)


# speedup vs baseline: 4.2138x; 1.0930x over previous
import functools

import jax
import jax.numpy as jnp
from jax import lax
from jax.experimental import pallas as pl
from jax.experimental.pallas import tpu as pltpu
from jax.experimental.pallas import tpu_sc as plsc

N_USERS = 50000
N_ITEMS = 50000
N_WARM = 40000
EMB = 64
FEAT = 256
N_LAYERS = 2
N_NEGS = 16
B = 4096
NNZ = 1200000
N_NODES = N_USERS + N_WARM
TAU = 0.2
LAMBDA_CL = 0.5
ALIGN_W = 0.1
REG_W = 1e-4
EPS = 1e-12

NW = 32
E_CH = 37504
K = 256
S = 1024
PAD_LEN = NW * E_CH + K
ND = EMB // 16

NA = B * (2 + N_NEGS)
NB = B * (1 + N_NEGS)
CA = NA // NW
CB = NB // NW
G = 128


def _zero_window(staging):
    def zb(i, _):
        for u in range(4):
            staging[pl.ds(i * 64 + u * 16, 16)] = jnp.zeros((16,), jnp.float32)
        return 0
    lax.fori_loop(0, S * EMB // 64, zb, 0)


def _spmm_body(row_hbm, col_hbm, val_hbm, src_hbm, out_hbm,
               cols_v, colsq_v, rows_v, vals_v, gbuf, staging, scal16, sem):
    wid = lax.axis_index("s") * 2 + lax.axis_index("c")
    start = wid * E_CH

    off0 = pl.multiple_of(jnp.maximum(start - 16, 0), 16)
    pltpu.sync_copy(row_hbm.at[pl.ds(off0, 16)], scal16)
    prev = jnp.where(wid == 0, jnp.int32(-1), scal16[...][15])
    off1 = pl.multiple_of(start + E_CH - 16, 16)
    pltpu.sync_copy(row_hbm.at[pl.ds(off1, 16)], scal16)
    last = jnp.minimum(scal16[...][15], jnp.int32(N_NODES - 1))

    _zero_window(staging)

    n_steps = ((PAD_LEN - start) // K + 1) + (last - prev + S - 1) // S + 2

    def step(_, state):
        pos, wbase, done = state

        def active(pos, wbase):
            wend = jnp.minimum(wbase + S, last + 1)
            pos_a = pl.multiple_of(pos, 128)
            cr = pltpu.async_copy(row_hbm.at[pl.ds(pos_a, K)], rows_v, sem)
            cv = pltpu.async_copy(val_hbm.at[pl.ds(pos_a, K)], vals_v, sem)
            cc = pltpu.async_copy(col_hbm.at[pl.ds(pos_a, K)], cols_v, sem)
            cr.wait(); cv.wait(); cc.wait()

            def shift_body(g, _):
                colsq_v[pl.ds(g * 16, 16)] = lax.shift_right_logical(
                    cols_v[pl.ds(g * 16, 16)], 1)
                return 0

            lax.fori_loop(0, K // 16, shift_body, 0)
            cps = [pltpu.async_copy(src_hbm.at[colsq_v.at[pl.ds(j * 128, 128)]],
                                    gbuf.at[pl.ds(j * 128, 128), :], sem)
                   for j in range(K // 128)]
            for cp in cps:
                cp.wait()

            def group_body(g, carry):
                rvec = rows_v[pl.ds(g * 16, 16)]
                vvec = vals_v[pl.ds(g * 16, 16)]
                hvec = jnp.bitwise_and(cols_v[pl.ds(g * 16, 16)], jnp.int32(1))
                m = (rvec >= wbase) & (rvec < wend)
                vmask = jnp.where(m, vvec, jnp.float32(0.0))
                offv = jnp.clip(rvec - wbase, 0, S - 1)
                for j in range(16):
                    off = offv[j]
                    vv = jnp.full((16,), vmask[j], jnp.float32)
                    e = g * 16 + j
                    base = pl.multiple_of(off * EMB, 16)
                    gb = pl.multiple_of(hvec[j] * EMB, 16)
                    for d in range(ND):
                        plsc.addupdate(staging.at[pl.ds(base + d * 16, 16)],
                                       vv * gbuf[e, pl.ds(gb + d * 16, 16)])
                return carry

            lax.fori_loop(0, K // 16, group_body, jnp.int32(0))
            batch_max = rows_v[pl.ds(K - 16, 16)][15]
            adv = batch_max >= wend
            new_pos = jnp.where(adv, pos_a, pos_a + K)

            @pl.when(adv & (wbase + S <= last + 1))
            def _():
                dst = pl.multiple_of(wbase * EMB, 64)
                pltpu.sync_copy(staging, out_hbm.at[pl.ds(dst, S * EMB)])
                _zero_window(staging)

            @pl.when(adv & (wbase + S > last + 1))
            def _():
                n = last + 1 - wbase
                offd = jnp.int32(0)
                for bit in reversed(range(10)):
                    sz = 1 << bit
                    take = (n & sz) != 0

                    @pl.when(take)
                    def _(offd=offd, sz=sz):
                        so = pl.multiple_of(offd * EMB, 64)
                        do = pl.multiple_of((wbase + offd) * EMB, 64)
                        pltpu.sync_copy(staging.at[pl.ds(so, sz * EMB)],
                                        out_hbm.at[pl.ds(do, sz * EMB)])

                    offd = offd + jnp.where(take, jnp.int32(sz), jnp.int32(0))

            new_wbase = jnp.where(adv, wbase + S, wbase)
            new_done = jnp.where(new_wbase > last, jnp.int32(1), jnp.int32(0))
            return new_pos, new_wbase, new_done

        return lax.cond(done == 0, active, lambda p, w: (p, w, done), pos, wbase)

    lax.fori_loop(0, n_steps, step,
                  (start, prev + 1, jnp.where(prev >= last, 1, 0).astype(jnp.int32)))


def _spmm(adj_row_pad, adj_col_pad, adj_val_pad, src_2d):
    mesh = plsc.VectorSubcoreMesh(core_axis_name="c", subcore_axis_name="s")
    kfn = functools.partial(
        pl.kernel,
        out_type=jax.ShapeDtypeStruct((N_NODES * EMB,), jnp.float32),
        mesh=mesh,
        scratch_types=[
            pltpu.VMEM((K,), jnp.int32),
            pltpu.VMEM((K,), jnp.int32),
            pltpu.VMEM((K,), jnp.int32),
            pltpu.VMEM((K,), jnp.float32),
            pltpu.VMEM((K, 128), jnp.float32),
            pltpu.VMEM((S * EMB,), jnp.float32),
            pltpu.VMEM((16,), jnp.int32),
            pltpu.SemaphoreType.DMA,
        ],
    )(_spmm_body)
    return kfn(adj_row_pad, adj_col_pad, adj_val_pad, src_2d)


def _gather_body(qa_hbm, qb_hbm, e0_hbm, e1_hbm, e2_hbm, fp_hbm,
                 outa_hbm, outb_hbm, idxb, g0, g1, g2, acc, sem):
    wid = lax.axis_index("s") * 2 + lax.axis_index("c")
    base_a = wid * CA
    base_b = wid * CB

    for t in range(CA // G):
        off = pl.multiple_of(base_a + t * G, 128)
        pltpu.sync_copy(qa_hbm.at[pl.ds(off, G)], idxb)
        c0 = pltpu.async_copy(e0_hbm.at[idxb], g0, sem)
        c1 = pltpu.async_copy(e1_hbm.at[idxb], g1, sem)
        c2 = pltpu.async_copy(e2_hbm.at[idxb], g2, sem)
        c0.wait(); c1.wait(); c2.wait()

        def avg_row(r, _):
            for u in range(8):
                sl = pl.ds(u * 16, 16)
                acc[r, sl] = (g0[r, sl] + g1[r, sl] + g2[r, sl]) * (1.0 / 3.0)
            return 0

        lax.fori_loop(0, G, avg_row, 0)
        pltpu.sync_copy(acc, outa_hbm.at[pl.ds(off, G), :])

    for t in range(CB // G):
        off = pl.multiple_of(base_b + t * G, 128)
        pltpu.sync_copy(qb_hbm.at[pl.ds(off, G)], idxb)
        pltpu.async_copy(fp_hbm.at[idxb], g0, sem).wait()
        pltpu.sync_copy(g0, outb_hbm.at[pl.ds(off, G), :])


def _gather_batch(qa, qb, e0_2d, e1_2d, e2_2d, fp_2d):
    mesh = plsc.VectorSubcoreMesh(core_axis_name="c", subcore_axis_name="s")
    kfn = functools.partial(
        pl.kernel,
        out_type=(jax.ShapeDtypeStruct((NA, 128), jnp.float32),
                  jax.ShapeDtypeStruct((NB, 128), jnp.float32)),
        mesh=mesh,
        scratch_types=[
            pltpu.VMEM((G,), jnp.int32),
            pltpu.VMEM((G, 128), jnp.float32),
            pltpu.VMEM((G, 128), jnp.float32),
            pltpu.VMEM((G, 128), jnp.float32),
            pltpu.VMEM((G, 128), jnp.float32),
            pltpu.SemaphoreType.DMA,
        ],
    )(_gather_body)
    return kfn(qa, qb, e0_2d, e1_2d, e2_2d, fp_2d)


_FP_BLK = 2000


def _feat_proj_body(feat_ref, w_ref, out_ref):
    x = feat_ref[...]
    w = w_ref[...]
    proj = lax.dot_general(x, w, (((1,), (1,)), ((), ())),
                           preferred_element_type=jnp.float32)
    nrm = jnp.sqrt(jnp.sum(proj * proj, axis=1, keepdims=True))
    out_ref[...] = proj / (nrm + EPS)


def _feat_proj(feat_warm, W):
    grid = N_WARM // _FP_BLK
    return pl.pallas_call(
        _feat_proj_body,
        grid=(grid,),
        in_specs=[
            pl.BlockSpec((_FP_BLK, FEAT), lambda i: (i, 0)),
            pl.BlockSpec((EMB, FEAT), lambda i: (0, 0)),
        ],
        out_specs=pl.BlockSpec((_FP_BLK, EMB), lambda i: (i, 0)),
        out_shape=jax.ShapeDtypeStruct((N_WARM, EMB), jnp.float32),
    )(feat_warm, W)


_L_BLK = 512


def _log_sigmoid(x):
    return jnp.minimum(x, 0.0) - jnp.log1p(jnp.exp(-jnp.abs(x)))


def _sel(x128, hf):
    return jnp.where(hf < 0.5, x128[:, :EMB], x128[:, EMB:])


def _loss_body(u_ref, pos_ref, negt_ref, pf_ref, nft_ref,
               hu_ref, hp_ref, hn_ref, hnf_ref, out_ref):
    i = pl.program_id(0)
    u = _sel(u_ref[...], hu_ref[...])
    pos = _sel(pos_ref[...], hp_ref[...])
    pf = _sel(pf_ref[...], hp_ref[...])

    pos_scores = jnp.sum(u * pos, axis=1)
    unrm = jnp.sqrt(jnp.sum(u * u, axis=1, keepdims=True))
    u_n = u / (unrm + EPS)
    pos_sim = jnp.sum(u_n * pf, axis=1) / TAU

    bpr_sum = jnp.float32(0.0)
    reg_neg = jnp.float32(0.0)
    m = pos_sim
    nf_sims = []
    for j in range(N_NEGS):
        ne = _sel(negt_ref[j], hn_ref[j])
        ns = jnp.sum(u * ne, axis=1)
        bpr_sum += jnp.sum(_log_sigmoid(pos_scores - ns))
        reg_neg += jnp.sum(ne * ne)
        nf = _sel(nft_ref[j], hnf_ref[j])
        nfs = jnp.sum(u_n * nf, axis=1) / TAU
        nf_sims.append(nfs)
        m = jnp.maximum(m, nfs)
    s = jnp.exp(pos_sim - m)
    for j in range(N_NEGS):
        s += jnp.exp(nf_sims[j] - m)
    cl_sum = jnp.sum(jnp.log(s) + m - pos_sim)

    pnrm = jnp.sqrt(jnp.sum(pos * pos, axis=1, keepdims=True))
    pos_n = pos / (pnrm + EPS)
    diff = pos_n - pf
    align_sum = jnp.sum(diff * diff)

    reg_sum = jnp.sum(u * u) + jnp.sum(pos * pos) + reg_neg

    contrib = (-bpr_sum / (B * N_NEGS)
               + (LAMBDA_CL / B) * cl_sum
               + (ALIGN_W / (B * EMB)) * align_sum
               + (REG_W / B) * reg_sum)
    contrib2d = jnp.full((1, 1), 0.0, jnp.float32) + contrib

    @pl.when(i == 0)
    def _():
        out_ref[...] = contrib2d

    @pl.when(i > 0)
    def _():
        out_ref[...] = out_ref[...] + contrib2d


def _loss(u_p, pos_p, neg_p, pf_p, nf_p, hu, hp, hn, hnf):
    grid = B // _L_BLK
    out = pl.pallas_call(
        _loss_body,
        grid=(grid,),
        in_specs=[
            pl.BlockSpec((_L_BLK, 128), lambda i: (i, 0)),
            pl.BlockSpec((_L_BLK, 128), lambda i: (i, 0)),
            pl.BlockSpec((N_NEGS, _L_BLK, 128), lambda i: (0, i, 0)),
            pl.BlockSpec((_L_BLK, 128), lambda i: (i, 0)),
            pl.BlockSpec((N_NEGS, _L_BLK, 128), lambda i: (0, i, 0)),
            pl.BlockSpec((_L_BLK, 1), lambda i: (i, 0)),
            pl.BlockSpec((_L_BLK, 1), lambda i: (i, 0)),
            pl.BlockSpec((N_NEGS, _L_BLK, 1), lambda i: (0, i, 0)),
            pl.BlockSpec((N_NEGS, _L_BLK, 1), lambda i: (0, i, 0)),
        ],
        out_specs=pl.BlockSpec((1, 1), lambda i: (0, 0)),
        out_shape=jax.ShapeDtypeStruct((1, 1), jnp.float32),
    )(u_p, pos_p, neg_p, pf_p, nf_p, hu, hp, hn, hnf)
    return out[0, 0]


def kernel(users, pos_items, neg_items, feat_all, user_emb, item_emb, W,
           adj_row, adj_col, adj_val, neg_feat_idx):
    all_emb = jnp.concatenate([user_emb, item_emb], axis=0)
    e0_2d = all_emb.reshape(N_NODES * EMB // 128, 128)

    pad = PAD_LEN - NNZ
    row_p = jnp.concatenate(
        [adj_row.astype(jnp.int32), jnp.full((pad,), N_NODES, jnp.int32)])
    col_p = jnp.concatenate(
        [adj_col.astype(jnp.int32),
         (jnp.arange(pad, dtype=jnp.int32) % N_NODES)])
    val_p = jnp.concatenate([adj_val, jnp.zeros((pad,), jnp.float32)])

    e1_2d = _spmm(row_p, col_p, val_p, e0_2d).reshape(N_NODES * EMB // 128, 128)
    e2_2d = _spmm(row_p, col_p, val_p, e1_2d).reshape(N_NODES * EMB // 128, 128)

    users32 = users.astype(jnp.int32)
    pos32 = pos_items.astype(jnp.int32)
    neg32 = neg_items.astype(jnp.int32)
    negf32 = neg_feat_idx.astype(jnp.int32)

    idx_a = jnp.concatenate(
        [users32, pos32 + N_USERS, (neg32.T + N_USERS).reshape(-1)])
    idx_b = jnp.concatenate([pos32, negf32.T.reshape(-1)])
    qa, ha = idx_a >> 1, (idx_a & 1).astype(jnp.float32)
    qb, hb = idx_b >> 1, (idx_b & 1).astype(jnp.float32)

    fp = _feat_proj(feat_all[:N_WARM], W)
    fp_2d = fp.reshape(N_WARM * EMB // 128, 128)

    outa, outb = _gather_batch(qa, qb, e0_2d, e1_2d, e2_2d, fp_2d)

    u_p = outa[:B]
    pos_p = outa[B:2 * B]
    neg_p = outa[2 * B:].reshape(N_NEGS, B, 128)
    pf_p = outb[:B]
    nf_p = outb[B:].reshape(N_NEGS, B, 128)

    hu = ha[:B].reshape(B, 1)
    hp = ha[B:2 * B].reshape(B, 1)
    hn = ha[2 * B:].reshape(N_NEGS, B, 1)
    hnf = hb[B:].reshape(N_NEGS, B, 1)

    return _loss(u_p, pos_p, neg_p, pf_p, nf_p, hu, hp, hn, hnf)


# speedup vs baseline: 4.5250x; 1.0739x over previous
import functools

import jax
import jax.numpy as jnp
from jax import lax
from jax.experimental import pallas as pl
from jax.experimental.pallas import tpu as pltpu
from jax.experimental.pallas import tpu_sc as plsc

N_USERS = 50000
N_ITEMS = 50000
N_WARM = 40000
EMB = 64
FEAT = 256
N_LAYERS = 2
N_NEGS = 16
B = 4096
NNZ = 1200000
N_NODES = N_USERS + N_WARM
TAU = 0.2
LAMBDA_CL = 0.5
ALIGN_W = 0.1
REG_W = 1e-4
EPS = 1e-12

NW = 32
E_CH = 37504
K = 256
S = 1024
PAD_LEN = NW * E_CH + K
ND = EMB // 16

NA = B * (2 + N_NEGS)
NB = B * (1 + N_NEGS)
CA = NA // NW
CB = NB // NW
G = 128


def _zero_window(staging):
    def zb(i, _):
        for u in range(4):
            staging[pl.ds(i * 64 + u * 16, 16)] = jnp.zeros((16,), jnp.float32)
        return 0
    lax.fori_loop(0, S * EMB // 64, zb, 0)


def _spmm_body(row_hbm, col_hbm, val_hbm, src_hbm, out_hbm,
               cols_v, colsq_v, rows_v, vals_v, gbuf, staging, scal16, sem):
    wid = lax.axis_index("s") * 2 + lax.axis_index("c")
    start = wid * E_CH

    off0 = pl.multiple_of(jnp.maximum(start - 16, 0), 16)
    pltpu.sync_copy(row_hbm.at[pl.ds(off0, 16)], scal16)
    prev = jnp.where(wid == 0, jnp.int32(-1), scal16[...][15])
    off1 = pl.multiple_of(start + E_CH - 16, 16)
    pltpu.sync_copy(row_hbm.at[pl.ds(off1, 16)], scal16)
    last = jnp.minimum(scal16[...][15], jnp.int32(N_NODES - 1))

    _zero_window(staging)

    n_steps = ((PAD_LEN - start) // K + 1) + (last - prev + S - 1) // S + 2

    def step(_, state):
        pos, wbase, done = state

        def active(pos, wbase):
            wend = jnp.minimum(wbase + S, last + 1)
            pos_a = pl.multiple_of(pos, 128)
            cr = pltpu.async_copy(row_hbm.at[pl.ds(pos_a, K)], rows_v, sem)
            cv = pltpu.async_copy(val_hbm.at[pl.ds(pos_a, K)], vals_v, sem)
            cc = pltpu.async_copy(col_hbm.at[pl.ds(pos_a, K)], cols_v, sem)
            cr.wait(); cv.wait(); cc.wait()

            def shift_body(g, _):
                colsq_v[pl.ds(g * 16, 16)] = lax.shift_right_logical(
                    cols_v[pl.ds(g * 16, 16)], 1)
                return 0

            lax.fori_loop(0, K // 16, shift_body, 0)
            cps = [pltpu.async_copy(src_hbm.at[colsq_v.at[pl.ds(j * 128, 128)]],
                                    gbuf.at[pl.ds(j * 128, 128), :], sem)
                   for j in range(K // 128)]

            def group_body(g, carry):
                rvec = rows_v[pl.ds(g * 16, 16)]
                vvec = vals_v[pl.ds(g * 16, 16)]
                hvec = jnp.bitwise_and(cols_v[pl.ds(g * 16, 16)], jnp.int32(1))
                m = (rvec >= wbase) & (rvec < wend)
                vmask = jnp.where(m, vvec, jnp.float32(0.0))
                offv = jnp.clip(rvec - wbase, 0, S - 1)
                for j in range(16):
                    off = offv[j]
                    vv = jnp.full((16,), vmask[j], jnp.float32)
                    e = g * 16 + j
                    base = pl.multiple_of(off * EMB, 16)
                    gb = pl.multiple_of(hvec[j] * EMB, 16)
                    for d in range(ND):
                        plsc.addupdate(staging.at[pl.ds(base + d * 16, 16)],
                                       vv * gbuf[e, pl.ds(gb + d * 16, 16)])
                return carry

            for h in range(K // 128):
                cps[h].wait()
                lax.fori_loop(h * 8, (h + 1) * 8, group_body, jnp.int32(0))

            batch_max = rows_v[pl.ds(K - 16, 16)][15]
            adv = batch_max >= wend
            new_pos = jnp.where(adv, pos_a, pos_a + K)

            @pl.when(adv & (wbase + S <= last + 1))
            def _():
                dst = pl.multiple_of(wbase * EMB, 64)
                pltpu.sync_copy(staging, out_hbm.at[pl.ds(dst, S * EMB)])
                _zero_window(staging)

            @pl.when(adv & (wbase + S > last + 1))
            def _():
                n = last + 1 - wbase
                offd = jnp.int32(0)
                for bit in reversed(range(10)):
                    sz = 1 << bit
                    take = (n & sz) != 0

                    @pl.when(take)
                    def _(offd=offd, sz=sz):
                        so = pl.multiple_of(offd * EMB, 64)
                        do = pl.multiple_of((wbase + offd) * EMB, 64)
                        pltpu.sync_copy(staging.at[pl.ds(so, sz * EMB)],
                                        out_hbm.at[pl.ds(do, sz * EMB)])

                    offd = offd + jnp.where(take, jnp.int32(sz), jnp.int32(0))

            new_wbase = jnp.where(adv, wbase + S, wbase)
            new_done = jnp.where(new_wbase > last, jnp.int32(1), jnp.int32(0))
            return new_pos, new_wbase, new_done

        return lax.cond(done == 0, active, lambda p, w: (p, w, done), pos, wbase)

    lax.fori_loop(0, n_steps, step,
                  (start, prev + 1, jnp.where(prev >= last, 1, 0).astype(jnp.int32)))


def _spmm(adj_row_pad, adj_col_pad, adj_val_pad, src_2d):
    mesh = plsc.VectorSubcoreMesh(core_axis_name="c", subcore_axis_name="s")
    kfn = functools.partial(
        pl.kernel,
        out_type=jax.ShapeDtypeStruct((N_NODES * EMB,), jnp.float32),
        mesh=mesh,
        scratch_types=[
            pltpu.VMEM((K,), jnp.int32),
            pltpu.VMEM((K,), jnp.int32),
            pltpu.VMEM((K,), jnp.int32),
            pltpu.VMEM((K,), jnp.float32),
            pltpu.VMEM((K, 128), jnp.float32),
            pltpu.VMEM((S * EMB,), jnp.float32),
            pltpu.VMEM((16,), jnp.int32),
            pltpu.SemaphoreType.DMA,
        ],
    )(_spmm_body)
    return kfn(adj_row_pad, adj_col_pad, adj_val_pad, src_2d)


def _gather_body(qa_hbm, qb_hbm, e0_hbm, e1_hbm, e2_hbm, fp_hbm,
                 outa_hbm, outb_hbm, idxb, g0, g1, g2, acc, sem):
    wid = lax.axis_index("s") * 2 + lax.axis_index("c")
    base_a = wid * CA
    base_b = wid * CB

    for t in range(CA // G):
        off = pl.multiple_of(base_a + t * G, 128)
        pltpu.sync_copy(qa_hbm.at[pl.ds(off, G)], idxb)
        c0 = pltpu.async_copy(e0_hbm.at[idxb], g0, sem)
        c1 = pltpu.async_copy(e1_hbm.at[idxb], g1, sem)
        c2 = pltpu.async_copy(e2_hbm.at[idxb], g2, sem)
        c0.wait(); c1.wait(); c2.wait()

        def avg_row(r, _):
            for u in range(8):
                sl = pl.ds(u * 16, 16)
                acc[r, sl] = (g0[r, sl] + g1[r, sl] + g2[r, sl]) * (1.0 / 3.0)
            return 0

        lax.fori_loop(0, G, avg_row, 0)
        pltpu.sync_copy(acc, outa_hbm.at[pl.ds(off, G), :])

    for t in range(CB // G):
        off = pl.multiple_of(base_b + t * G, 128)
        pltpu.sync_copy(qb_hbm.at[pl.ds(off, G)], idxb)
        pltpu.async_copy(fp_hbm.at[idxb], g0, sem).wait()
        pltpu.sync_copy(g0, outb_hbm.at[pl.ds(off, G), :])


def _gather_batch(qa, qb, e0_2d, e1_2d, e2_2d, fp_2d):
    mesh = plsc.VectorSubcoreMesh(core_axis_name="c", subcore_axis_name="s")
    kfn = functools.partial(
        pl.kernel,
        out_type=(jax.ShapeDtypeStruct((NA, 128), jnp.float32),
                  jax.ShapeDtypeStruct((NB, 128), jnp.float32)),
        mesh=mesh,
        scratch_types=[
            pltpu.VMEM((G,), jnp.int32),
            pltpu.VMEM((G, 128), jnp.float32),
            pltpu.VMEM((G, 128), jnp.float32),
            pltpu.VMEM((G, 128), jnp.float32),
            pltpu.VMEM((G, 128), jnp.float32),
            pltpu.SemaphoreType.DMA,
        ],
    )(_gather_body)
    return kfn(qa, qb, e0_2d, e1_2d, e2_2d, fp_2d)


_FP_BLK = 2000


def _feat_proj_body(feat_ref, w_ref, out_ref):
    x = feat_ref[...]
    w = w_ref[...]
    proj = lax.dot_general(x, w, (((1,), (1,)), ((), ())),
                           preferred_element_type=jnp.float32)
    nrm = jnp.sqrt(jnp.sum(proj * proj, axis=1, keepdims=True))
    out_ref[...] = proj / (nrm + EPS)


def _feat_proj(feat_warm, W):
    grid = N_WARM // _FP_BLK
    return pl.pallas_call(
        _feat_proj_body,
        grid=(grid,),
        in_specs=[
            pl.BlockSpec((_FP_BLK, FEAT), lambda i: (i, 0)),
            pl.BlockSpec((EMB, FEAT), lambda i: (0, 0)),
        ],
        out_specs=pl.BlockSpec((_FP_BLK, EMB), lambda i: (i, 0)),
        out_shape=jax.ShapeDtypeStruct((N_WARM, EMB), jnp.float32),
    )(feat_warm, W)


_L_BLK = 512


def _log_sigmoid(x):
    return jnp.minimum(x, 0.0) - jnp.log1p(jnp.exp(-jnp.abs(x)))


def _sel(x128, hf):
    return jnp.where(hf < 0.5, x128[:, :EMB], x128[:, EMB:])


def _loss_body(u_ref, pos_ref, negt_ref, pf_ref, nft_ref,
               hu_ref, hp_ref, hn_ref, hnf_ref, out_ref):
    i = pl.program_id(0)
    u = _sel(u_ref[...], hu_ref[...])
    pos = _sel(pos_ref[...], hp_ref[...])
    pf = _sel(pf_ref[...], hp_ref[...])

    pos_scores = jnp.sum(u * pos, axis=1)
    unrm = jnp.sqrt(jnp.sum(u * u, axis=1, keepdims=True))
    u_n = u / (unrm + EPS)
    pos_sim = jnp.sum(u_n * pf, axis=1) / TAU

    bpr_sum = jnp.float32(0.0)
    reg_neg = jnp.float32(0.0)
    m = pos_sim
    nf_sims = []
    for j in range(N_NEGS):
        ne = _sel(negt_ref[j], hn_ref[j])
        ns = jnp.sum(u * ne, axis=1)
        bpr_sum += jnp.sum(_log_sigmoid(pos_scores - ns))
        reg_neg += jnp.sum(ne * ne)
        nf = _sel(nft_ref[j], hnf_ref[j])
        nfs = jnp.sum(u_n * nf, axis=1) / TAU
        nf_sims.append(nfs)
        m = jnp.maximum(m, nfs)
    s = jnp.exp(pos_sim - m)
    for j in range(N_NEGS):
        s += jnp.exp(nf_sims[j] - m)
    cl_sum = jnp.sum(jnp.log(s) + m - pos_sim)

    pnrm = jnp.sqrt(jnp.sum(pos * pos, axis=1, keepdims=True))
    pos_n = pos / (pnrm + EPS)
    diff = pos_n - pf
    align_sum = jnp.sum(diff * diff)

    reg_sum = jnp.sum(u * u) + jnp.sum(pos * pos) + reg_neg

    contrib = (-bpr_sum / (B * N_NEGS)
               + (LAMBDA_CL / B) * cl_sum
               + (ALIGN_W / (B * EMB)) * align_sum
               + (REG_W / B) * reg_sum)
    contrib2d = jnp.full((1, 1), 0.0, jnp.float32) + contrib

    @pl.when(i == 0)
    def _():
        out_ref[...] = contrib2d

    @pl.when(i > 0)
    def _():
        out_ref[...] = out_ref[...] + contrib2d


def _loss(u_p, pos_p, neg_p, pf_p, nf_p, hu, hp, hn, hnf):
    grid = B // _L_BLK
    out = pl.pallas_call(
        _loss_body,
        grid=(grid,),
        in_specs=[
            pl.BlockSpec((_L_BLK, 128), lambda i: (i, 0)),
            pl.BlockSpec((_L_BLK, 128), lambda i: (i, 0)),
            pl.BlockSpec((N_NEGS, _L_BLK, 128), lambda i: (0, i, 0)),
            pl.BlockSpec((_L_BLK, 128), lambda i: (i, 0)),
            pl.BlockSpec((N_NEGS, _L_BLK, 128), lambda i: (0, i, 0)),
            pl.BlockSpec((_L_BLK, 1), lambda i: (i, 0)),
            pl.BlockSpec((_L_BLK, 1), lambda i: (i, 0)),
            pl.BlockSpec((N_NEGS, _L_BLK, 1), lambda i: (0, i, 0)),
            pl.BlockSpec((N_NEGS, _L_BLK, 1), lambda i: (0, i, 0)),
        ],
        out_specs=pl.BlockSpec((1, 1), lambda i: (0, 0)),
        out_shape=jax.ShapeDtypeStruct((1, 1), jnp.float32),
    )(u_p, pos_p, neg_p, pf_p, nf_p, hu, hp, hn, hnf)
    return out[0, 0]


def kernel(users, pos_items, neg_items, feat_all, user_emb, item_emb, W,
           adj_row, adj_col, adj_val, neg_feat_idx):
    all_emb = jnp.concatenate([user_emb, item_emb], axis=0)
    e0_2d = all_emb.reshape(N_NODES * EMB // 128, 128)

    pad = PAD_LEN - NNZ
    row_p = jnp.concatenate(
        [adj_row.astype(jnp.int32), jnp.full((pad,), N_NODES, jnp.int32)])
    col_p = jnp.concatenate(
        [adj_col.astype(jnp.int32),
         (jnp.arange(pad, dtype=jnp.int32) % N_NODES)])
    val_p = jnp.concatenate([adj_val, jnp.zeros((pad,), jnp.float32)])

    e1_2d = _spmm(row_p, col_p, val_p, e0_2d).reshape(N_NODES * EMB // 128, 128)
    e2_2d = _spmm(row_p, col_p, val_p, e1_2d).reshape(N_NODES * EMB // 128, 128)

    users32 = users.astype(jnp.int32)
    pos32 = pos_items.astype(jnp.int32)
    neg32 = neg_items.astype(jnp.int32)
    negf32 = neg_feat_idx.astype(jnp.int32)

    idx_a = jnp.concatenate(
        [users32, pos32 + N_USERS, (neg32.T + N_USERS).reshape(-1)])
    idx_b = jnp.concatenate([pos32, negf32.T.reshape(-1)])
    qa, ha = idx_a >> 1, (idx_a & 1).astype(jnp.float32)
    qb, hb = idx_b >> 1, (idx_b & 1).astype(jnp.float32)

    fp = _feat_proj(feat_all[:N_WARM], W)
    fp_2d = fp.reshape(N_WARM * EMB // 128, 128)

    outa, outb = _gather_batch(qa, qb, e0_2d, e1_2d, e2_2d, fp_2d)

    u_p = outa[:B]
    pos_p = outa[B:2 * B]
    neg_p = outa[2 * B:].reshape(N_NEGS, B, 128)
    pf_p = outb[:B]
    nf_p = outb[B:].reshape(N_NEGS, B, 128)

    hu = ha[:B].reshape(B, 1)
    hp = ha[B:2 * B].reshape(B, 1)
    hn = ha[2 * B:].reshape(N_NEGS, B, 1)
    hnf = hb[B:].reshape(N_NEGS, B, 1)

    return _loss(u_p, pos_p, neg_p, pf_p, nf_p, hu, hp, hn, hnf)


# speedup vs baseline: 4.6351x; 1.0243x over previous
import functools

import jax
import jax.numpy as jnp
from jax import lax
from jax.experimental import pallas as pl
from jax.experimental.pallas import tpu as pltpu
from jax.experimental.pallas import tpu_sc as plsc

N_USERS = 50000
N_ITEMS = 50000
N_WARM = 40000
EMB = 64
FEAT = 256
N_LAYERS = 2
N_NEGS = 16
B = 4096
NNZ = 1200000
N_NODES = N_USERS + N_WARM
TAU = 0.2
LAMBDA_CL = 0.5
ALIGN_W = 0.1
REG_W = 1e-4
EPS = 1e-12

NW = 32
E_CH = 37504
K = 256
S = 1024
PAD_LEN = NW * E_CH + K
ND = EMB // 16

NA = B * (2 + N_NEGS)
NB = B * (1 + N_NEGS)
CA = NA // NW
CB = NB // NW
G = 128


def _zero_window(staging):
    def zb(i, _):
        for u in range(4):
            staging[pl.ds(i * 64 + u * 16, 16)] = jnp.zeros((16,), jnp.float32)
        return 0
    lax.fori_loop(0, S * EMB // 64, zb, 0)


def _spmm_body(row_hbm, col_hbm, val_hbm, src_hbm, out_hbm,
               cols_v, colsq_v, rows_v, vals_v, gbuf, staging, scal16, sem):
    wid = lax.axis_index("s") * 2 + lax.axis_index("c")
    start = wid * E_CH

    off0 = pl.multiple_of(jnp.maximum(start - 16, 0), 16)
    pltpu.sync_copy(row_hbm.at[pl.ds(off0, 16)], scal16)
    prev = jnp.where(wid == 0, jnp.int32(-1), scal16[...][15])
    off1 = pl.multiple_of(start + E_CH - 16, 16)
    pltpu.sync_copy(row_hbm.at[pl.ds(off1, 16)], scal16)
    last = jnp.minimum(scal16[...][15], jnp.int32(N_NODES - 1))

    _zero_window(staging)

    def step(_, state):
        pos, wbase, done = state
        nd = done == 0
        wend = jnp.minimum(wbase + S, last + 1)
        pos_a = pl.multiple_of(pos, 128)

        @pl.when(nd)
        def _():
            cr = pltpu.async_copy(row_hbm.at[pl.ds(pos_a, K)], rows_v, sem)
            cv = pltpu.async_copy(val_hbm.at[pl.ds(pos_a, K)], vals_v, sem)
            cc = pltpu.async_copy(col_hbm.at[pl.ds(pos_a, K)], cols_v, sem)
            cr.wait(); cv.wait(); cc.wait()

            def shift_body(g, _):
                colsq_v[pl.ds(g * 16, 16)] = lax.shift_right_logical(
                    cols_v[pl.ds(g * 16, 16)], 1)
                return 0

            lax.fori_loop(0, K // 16, shift_body, 0)
            cps = [pltpu.async_copy(src_hbm.at[colsq_v.at[pl.ds(j * 128, 128)]],
                                    gbuf.at[pl.ds(j * 128, 128), :], sem)
                   for j in range(K // 128)]

            def group_body(g, carry):
                rvec = rows_v[pl.ds(g * 16, 16)]
                vvec = vals_v[pl.ds(g * 16, 16)]
                hvec = jnp.bitwise_and(cols_v[pl.ds(g * 16, 16)], jnp.int32(1))
                m = (rvec >= wbase) & (rvec < wend)
                vmask = jnp.where(m, vvec, jnp.float32(0.0))
                offv = jnp.clip(rvec - wbase, 0, S - 1) * EMB
                gbv = hvec * EMB
                for j in range(16):
                    vv = jnp.full((16,), vmask[j], jnp.float32)
                    e = g * 16 + j
                    base = pl.multiple_of(offv[j], 16)
                    gb = pl.multiple_of(gbv[j], 16)
                    for d in range(ND):
                        plsc.addupdate(staging.at[pl.ds(base + d * 16, 16)],
                                       vv * gbuf[e, pl.ds(gb + d * 16, 16)])
                return carry

            for h in range(K // 128):
                cps[h].wait()
                lax.fori_loop(h * 8, (h + 1) * 8, group_body, jnp.int32(0))

        batch_max = rows_v[pl.ds(K - 16, 16)][15]
        adv = batch_max >= wend

        @pl.when(nd & adv & (wbase + S <= last + 1))
        def _():
            dst = pl.multiple_of(wbase * EMB, 64)
            pltpu.async_copy(staging, out_hbm.at[pl.ds(dst, S * EMB)],
                             sem).wait()
            _zero_window(staging)

        @pl.when(nd & adv & (wbase + S > last + 1))
        def _():
            n = last + 1 - wbase
            offd = jnp.int32(0)
            for bit in reversed(range(10)):
                sz = 1 << bit
                take = (n & sz) != 0

                @pl.when(take)
                def _(offd=offd, sz=sz):
                    so = pl.multiple_of(offd * EMB, 64)
                    do = pl.multiple_of((wbase + offd) * EMB, 64)
                    pltpu.async_copy(staging.at[pl.ds(so, sz * EMB)],
                                     out_hbm.at[pl.ds(do, sz * EMB)],
                                     sem).wait()

                offd = offd + jnp.where(take, jnp.int32(sz), jnp.int32(0))

        new_pos = jnp.where(nd & jnp.logical_not(adv), pos_a + K, pos_a)
        new_wbase = jnp.where(nd & adv, wbase + S, wbase)
        new_done = jnp.where(new_wbase > last, jnp.int32(1), done)
        return new_pos, new_wbase, new_done

    T_STEPS = (E_CH + K - 1) // K + 1 + (N_NODES + S - 1) // S + 4
    lax.fori_loop(0, T_STEPS, step,
                  (start, prev + 1,
                   jnp.where(prev >= last, 1, 0).astype(jnp.int32)))


def _spmm(adj_row_pad, adj_col_pad, adj_val_pad, src_2d):
    mesh = plsc.VectorSubcoreMesh(core_axis_name="c", subcore_axis_name="s")
    kfn = functools.partial(
        pl.kernel,
        out_type=jax.ShapeDtypeStruct((N_NODES * EMB,), jnp.float32),
        mesh=mesh,
        scratch_types=[
            pltpu.VMEM((K,), jnp.int32),
            pltpu.VMEM((K,), jnp.int32),
            pltpu.VMEM((K,), jnp.int32),
            pltpu.VMEM((K,), jnp.float32),
            pltpu.VMEM((K, 128), jnp.float32),
            pltpu.VMEM((S * EMB,), jnp.float32),
            pltpu.VMEM((16,), jnp.int32),
            pltpu.SemaphoreType.DMA,
        ],
    )(_spmm_body)
    return kfn(adj_row_pad, adj_col_pad, adj_val_pad, src_2d)


def _gather_body(qa_hbm, qb_hbm, e0_hbm, e1_hbm, e2_hbm, fp_hbm,
                 outa_hbm, outb_hbm, idxb, g0, g1, g2, acc, sem):
    wid = lax.axis_index("s") * 2 + lax.axis_index("c")
    base_a = wid * CA
    base_b = wid * CB

    for t in range(CA // G):
        off = pl.multiple_of(base_a + t * G, 128)
        pltpu.sync_copy(qa_hbm.at[pl.ds(off, G)], idxb)
        c0 = pltpu.async_copy(e0_hbm.at[idxb], g0, sem)
        c1 = pltpu.async_copy(e1_hbm.at[idxb], g1, sem)
        c2 = pltpu.async_copy(e2_hbm.at[idxb], g2, sem)
        c0.wait(); c1.wait(); c2.wait()

        def avg_row(r, _):
            for u in range(8):
                sl = pl.ds(u * 16, 16)
                acc[r, sl] = (g0[r, sl] + g1[r, sl] + g2[r, sl]) * (1.0 / 3.0)
            return 0

        lax.fori_loop(0, G, avg_row, 0)
        pltpu.sync_copy(acc, outa_hbm.at[pl.ds(off, G), :])

    for t in range(CB // G):
        off = pl.multiple_of(base_b + t * G, 128)
        pltpu.sync_copy(qb_hbm.at[pl.ds(off, G)], idxb)
        pltpu.async_copy(fp_hbm.at[idxb], g0, sem).wait()
        pltpu.sync_copy(g0, outb_hbm.at[pl.ds(off, G), :])


def _gather_batch(qa, qb, e0_2d, e1_2d, e2_2d, fp_2d):
    mesh = plsc.VectorSubcoreMesh(core_axis_name="c", subcore_axis_name="s")
    kfn = functools.partial(
        pl.kernel,
        out_type=(jax.ShapeDtypeStruct((NA, 128), jnp.float32),
                  jax.ShapeDtypeStruct((NB, 128), jnp.float32)),
        mesh=mesh,
        scratch_types=[
            pltpu.VMEM((G,), jnp.int32),
            pltpu.VMEM((G, 128), jnp.float32),
            pltpu.VMEM((G, 128), jnp.float32),
            pltpu.VMEM((G, 128), jnp.float32),
            pltpu.VMEM((G, 128), jnp.float32),
            pltpu.SemaphoreType.DMA,
        ],
    )(_gather_body)
    return kfn(qa, qb, e0_2d, e1_2d, e2_2d, fp_2d)


_FP_BLK = 2000


def _feat_proj_body(feat_ref, w_ref, out_ref):
    x = feat_ref[...]
    w = w_ref[...]
    proj = lax.dot_general(x, w, (((1,), (1,)), ((), ())),
                           preferred_element_type=jnp.float32)
    nrm = jnp.sqrt(jnp.sum(proj * proj, axis=1, keepdims=True))
    out_ref[...] = proj / (nrm + EPS)


def _feat_proj(feat_warm, W):
    grid = N_WARM // _FP_BLK
    return pl.pallas_call(
        _feat_proj_body,
        grid=(grid,),
        in_specs=[
            pl.BlockSpec((_FP_BLK, FEAT), lambda i: (i, 0)),
            pl.BlockSpec((EMB, FEAT), lambda i: (0, 0)),
        ],
        out_specs=pl.BlockSpec((_FP_BLK, EMB), lambda i: (i, 0)),
        out_shape=jax.ShapeDtypeStruct((N_WARM, EMB), jnp.float32),
    )(feat_warm, W)


_L_BLK = 512


def _log_sigmoid(x):
    return jnp.minimum(x, 0.0) - jnp.log1p(jnp.exp(-jnp.abs(x)))


def _sel(x128, hf):
    return jnp.where(hf < 0.5, x128[:, :EMB], x128[:, EMB:])


def _loss_body(u_ref, pos_ref, negt_ref, pf_ref, nft_ref,
               hu_ref, hp_ref, hn_ref, hnf_ref, out_ref):
    i = pl.program_id(0)
    u = _sel(u_ref[...], hu_ref[...])
    pos = _sel(pos_ref[...], hp_ref[...])
    pf = _sel(pf_ref[...], hp_ref[...])

    pos_scores = jnp.sum(u * pos, axis=1)
    unrm = jnp.sqrt(jnp.sum(u * u, axis=1, keepdims=True))
    u_n = u / (unrm + EPS)
    pos_sim = jnp.sum(u_n * pf, axis=1) / TAU

    bpr_sum = jnp.float32(0.0)
    reg_neg = jnp.float32(0.0)
    m = pos_sim
    nf_sims = []
    for j in range(N_NEGS):
        ne = _sel(negt_ref[j], hn_ref[j])
        ns = jnp.sum(u * ne, axis=1)
        bpr_sum += jnp.sum(_log_sigmoid(pos_scores - ns))
        reg_neg += jnp.sum(ne * ne)
        nf = _sel(nft_ref[j], hnf_ref[j])
        nfs = jnp.sum(u_n * nf, axis=1) / TAU
        nf_sims.append(nfs)
        m = jnp.maximum(m, nfs)
    s = jnp.exp(pos_sim - m)
    for j in range(N_NEGS):
        s += jnp.exp(nf_sims[j] - m)
    cl_sum = jnp.sum(jnp.log(s) + m - pos_sim)

    pnrm = jnp.sqrt(jnp.sum(pos * pos, axis=1, keepdims=True))
    pos_n = pos / (pnrm + EPS)
    diff = pos_n - pf
    align_sum = jnp.sum(diff * diff)

    reg_sum = jnp.sum(u * u) + jnp.sum(pos * pos) + reg_neg

    contrib = (-bpr_sum / (B * N_NEGS)
               + (LAMBDA_CL / B) * cl_sum
               + (ALIGN_W / (B * EMB)) * align_sum
               + (REG_W / B) * reg_sum)
    contrib2d = jnp.full((1, 1), 0.0, jnp.float32) + contrib

    @pl.when(i == 0)
    def _():
        out_ref[...] = contrib2d

    @pl.when(i > 0)
    def _():
        out_ref[...] = out_ref[...] + contrib2d


def _loss(u_p, pos_p, neg_p, pf_p, nf_p, hu, hp, hn, hnf):
    grid = B // _L_BLK
    out = pl.pallas_call(
        _loss_body,
        grid=(grid,),
        in_specs=[
            pl.BlockSpec((_L_BLK, 128), lambda i: (i, 0)),
            pl.BlockSpec((_L_BLK, 128), lambda i: (i, 0)),
            pl.BlockSpec((N_NEGS, _L_BLK, 128), lambda i: (0, i, 0)),
            pl.BlockSpec((_L_BLK, 128), lambda i: (i, 0)),
            pl.BlockSpec((N_NEGS, _L_BLK, 128), lambda i: (0, i, 0)),
            pl.BlockSpec((_L_BLK, 1), lambda i: (i, 0)),
            pl.BlockSpec((_L_BLK, 1), lambda i: (i, 0)),
            pl.BlockSpec((N_NEGS, _L_BLK, 1), lambda i: (0, i, 0)),
            pl.BlockSpec((N_NEGS, _L_BLK, 1), lambda i: (0, i, 0)),
        ],
        out_specs=pl.BlockSpec((1, 1), lambda i: (0, 0)),
        out_shape=jax.ShapeDtypeStruct((1, 1), jnp.float32),
    )(u_p, pos_p, neg_p, pf_p, nf_p, hu, hp, hn, hnf)
    return out[0, 0]


def kernel(users, pos_items, neg_items, feat_all, user_emb, item_emb, W,
           adj_row, adj_col, adj_val, neg_feat_idx):
    all_emb = jnp.concatenate([user_emb, item_emb], axis=0)
    e0_2d = all_emb.reshape(N_NODES * EMB // 128, 128)

    pad = PAD_LEN - NNZ
    row_p = jnp.concatenate(
        [adj_row.astype(jnp.int32), jnp.full((pad,), N_NODES, jnp.int32)])
    col_p = jnp.concatenate(
        [adj_col.astype(jnp.int32),
         (jnp.arange(pad, dtype=jnp.int32) % N_NODES)])
    val_p = jnp.concatenate([adj_val, jnp.zeros((pad,), jnp.float32)])

    e1_2d = _spmm(row_p, col_p, val_p, e0_2d).reshape(N_NODES * EMB // 128, 128)
    e2_2d = _spmm(row_p, col_p, val_p, e1_2d).reshape(N_NODES * EMB // 128, 128)

    users32 = users.astype(jnp.int32)
    pos32 = pos_items.astype(jnp.int32)
    neg32 = neg_items.astype(jnp.int32)
    negf32 = neg_feat_idx.astype(jnp.int32)

    idx_a = jnp.concatenate(
        [users32, pos32 + N_USERS, (neg32.T + N_USERS).reshape(-1)])
    idx_b = jnp.concatenate([pos32, negf32.T.reshape(-1)])
    qa, ha = idx_a >> 1, (idx_a & 1).astype(jnp.float32)
    qb, hb = idx_b >> 1, (idx_b & 1).astype(jnp.float32)

    fp = _feat_proj(feat_all[:N_WARM], W)
    fp_2d = fp.reshape(N_WARM * EMB // 128, 128)

    outa, outb = _gather_batch(qa, qb, e0_2d, e1_2d, e2_2d, fp_2d)

    u_p = outa[:B]
    pos_p = outa[B:2 * B]
    neg_p = outa[2 * B:].reshape(N_NEGS, B, 128)
    pf_p = outb[:B]
    nf_p = outb[B:].reshape(N_NEGS, B, 128)

    hu = ha[:B].reshape(B, 1)
    hp = ha[B:2 * B].reshape(B, 1)
    hn = ha[2 * B:].reshape(N_NEGS, B, 1)
    hnf = hb[B:].reshape(N_NEGS, B, 1)

    return _loss(u_p, pos_p, neg_p, pf_p, nf_p, hu, hp, hn, hnf)


# speedup vs baseline: 5.0105x; 1.0810x over previous
import functools

import jax
import jax.numpy as jnp
from jax import lax
from jax.experimental import pallas as pl
from jax.experimental.pallas import tpu as pltpu
from jax.experimental.pallas import tpu_sc as plsc

N_USERS = 50000
N_ITEMS = 50000
N_WARM = 40000
EMB = 64
FEAT = 256
N_LAYERS = 2
N_NEGS = 16
B = 4096
NNZ = 1200000
N_NODES = N_USERS + N_WARM
TAU = 0.2
LAMBDA_CL = 0.5
ALIGN_W = 0.1
REG_W = 1e-4
EPS = 1e-12

NW = 32
E_CH = 37504
K = 512
S = 768
PAD_LEN = NW * E_CH + K
ND = EMB // 16

NA = B * (2 + N_NEGS)
NB = B * (1 + N_NEGS)
CA = NA // NW
CB = NB // NW
G = 128


def _zero_window(staging):
    def zb(i, _):
        for u in range(4):
            staging[pl.ds(i * 64 + u * 16, 16)] = jnp.zeros((16,), jnp.float32)
        return 0
    lax.fori_loop(0, S * EMB // 64, zb, 0)


def _spmm_body(row_hbm, col_hbm, val_hbm, src_hbm, out_hbm,
               cols_v, colsq_v, rows_v, vals_v, gbuf, staging, scal16, sem):
    wid = lax.axis_index("s") * 2 + lax.axis_index("c")
    start = wid * E_CH

    off0 = pl.multiple_of(jnp.maximum(start - 16, 0), 16)
    pltpu.sync_copy(row_hbm.at[pl.ds(off0, 16)], scal16)
    prev = jnp.where(wid == 0, jnp.int32(-1), scal16[...][15])
    off1 = pl.multiple_of(start + E_CH - 16, 16)
    pltpu.sync_copy(row_hbm.at[pl.ds(off1, 16)], scal16)
    last = jnp.minimum(scal16[...][15], jnp.int32(N_NODES - 1))

    _zero_window(staging)

    def step(_, state):
        pos, wbase, done = state
        nd = done == 0
        wend = jnp.minimum(wbase + S, last + 1)
        pos_a = pl.multiple_of(pos, 128)

        @pl.when(nd)
        def _():
            cr = pltpu.async_copy(row_hbm.at[pl.ds(pos_a, K)], rows_v, sem)
            cv = pltpu.async_copy(val_hbm.at[pl.ds(pos_a, K)], vals_v, sem)
            cc = pltpu.async_copy(col_hbm.at[pl.ds(pos_a, K)], cols_v, sem)
            cr.wait(); cv.wait(); cc.wait()

            def shift_body(g, _):
                colsq_v[pl.ds(g * 16, 16)] = lax.shift_right_logical(
                    cols_v[pl.ds(g * 16, 16)], 1)
                return 0

            lax.fori_loop(0, K // 16, shift_body, 0)
            cps = [pltpu.async_copy(src_hbm.at[colsq_v.at[pl.ds(j * 128, 128)]],
                                    gbuf.at[pl.ds(j * 128, 128), :], sem)
                   for j in range(K // 128)]

            def group_body(g, carry):
                rvec = rows_v[pl.ds(g * 16, 16)]
                vvec = vals_v[pl.ds(g * 16, 16)]
                hvec = jnp.bitwise_and(cols_v[pl.ds(g * 16, 16)], jnp.int32(1))
                m = (rvec >= wbase) & (rvec < wend)
                vmask = jnp.where(m, vvec, jnp.float32(0.0))
                offv = jnp.clip(rvec - wbase, 0, S - 1) * EMB
                gbv = hvec * EMB
                for j in range(16):
                    vv = jnp.full((16,), vmask[j], jnp.float32)
                    e = g * 16 + j
                    base = pl.multiple_of(offv[j], 16)
                    gb = pl.multiple_of(gbv[j], 16)
                    for d in range(ND):
                        plsc.addupdate(staging.at[pl.ds(base + d * 16, 16)],
                                       vv * gbuf[e, pl.ds(gb + d * 16, 16)])
                return carry

            for h in range(K // 128):
                cps[h].wait()
                lax.fori_loop(h * 8, (h + 1) * 8, group_body, jnp.int32(0))

        batch_max = rows_v[pl.ds(K - 16, 16)][15]
        adv = batch_max >= wend

        @pl.when(nd & adv & (wbase + S <= last + 1))
        def _():
            dst = pl.multiple_of(wbase * EMB, 64)
            pltpu.async_copy(staging, out_hbm.at[pl.ds(dst, S * EMB)],
                             sem).wait()
            _zero_window(staging)

        @pl.when(nd & adv & (wbase + S > last + 1))
        def _():
            n = last + 1 - wbase
            offd = jnp.int32(0)
            for bit in reversed(range(10)):
                sz = 1 << bit
                take = (n & sz) != 0

                @pl.when(take)
                def _(offd=offd, sz=sz):
                    so = pl.multiple_of(offd * EMB, 64)
                    do = pl.multiple_of((wbase + offd) * EMB, 64)
                    pltpu.async_copy(staging.at[pl.ds(so, sz * EMB)],
                                     out_hbm.at[pl.ds(do, sz * EMB)],
                                     sem).wait()

                offd = offd + jnp.where(take, jnp.int32(sz), jnp.int32(0))

        new_pos = jnp.where(nd & jnp.logical_not(adv), pos_a + K, pos_a)
        new_wbase = jnp.where(nd & adv, wbase + S, wbase)
        new_done = jnp.where(new_wbase > last, jnp.int32(1), done)
        return new_pos, new_wbase, new_done

    T_STEPS = (E_CH + K - 1) // K + 1 + (N_NODES + S - 1) // S + 4
    lax.fori_loop(0, T_STEPS, step,
                  (start, prev + 1,
                   jnp.where(prev >= last, 1, 0).astype(jnp.int32)))


def _spmm(adj_row_pad, adj_col_pad, adj_val_pad, src_2d):
    mesh = plsc.VectorSubcoreMesh(core_axis_name="c", subcore_axis_name="s")
    kfn = functools.partial(
        pl.kernel,
        out_type=jax.ShapeDtypeStruct((N_NODES * EMB,), jnp.float32),
        mesh=mesh,
        scratch_types=[
            pltpu.VMEM((K,), jnp.int32),
            pltpu.VMEM((K,), jnp.int32),
            pltpu.VMEM((K,), jnp.int32),
            pltpu.VMEM((K,), jnp.float32),
            pltpu.VMEM((K, 128), jnp.float32),
            pltpu.VMEM((S * EMB,), jnp.float32),
            pltpu.VMEM((16,), jnp.int32),
            pltpu.SemaphoreType.DMA,
        ],
    )(_spmm_body)
    return kfn(adj_row_pad, adj_col_pad, adj_val_pad, src_2d)


def _gather_body(qa_hbm, qb_hbm, e0_hbm, e1_hbm, e2_hbm, fp_hbm,
                 outa_hbm, outb_hbm, idxb, g0, g1, g2, acc, sem):
    wid = lax.axis_index("s") * 2 + lax.axis_index("c")
    base_a = wid * CA
    base_b = wid * CB

    for t in range(CA // G):
        off = pl.multiple_of(base_a + t * G, 128)
        pltpu.sync_copy(qa_hbm.at[pl.ds(off, G)], idxb)
        c0 = pltpu.async_copy(e0_hbm.at[idxb], g0, sem)
        c1 = pltpu.async_copy(e1_hbm.at[idxb], g1, sem)
        c2 = pltpu.async_copy(e2_hbm.at[idxb], g2, sem)
        c0.wait(); c1.wait(); c2.wait()

        def avg_row(r, _):
            for u in range(8):
                sl = pl.ds(u * 16, 16)
                acc[r, sl] = (g0[r, sl] + g1[r, sl] + g2[r, sl]) * (1.0 / 3.0)
            return 0

        lax.fori_loop(0, G, avg_row, 0)
        pltpu.sync_copy(acc, outa_hbm.at[pl.ds(off, G), :])

    for t in range(CB // G):
        off = pl.multiple_of(base_b + t * G, 128)
        pltpu.sync_copy(qb_hbm.at[pl.ds(off, G)], idxb)
        pltpu.async_copy(fp_hbm.at[idxb], g0, sem).wait()
        pltpu.sync_copy(g0, outb_hbm.at[pl.ds(off, G), :])


def _gather_batch(qa, qb, e0_2d, e1_2d, e2_2d, fp_2d):
    mesh = plsc.VectorSubcoreMesh(core_axis_name="c", subcore_axis_name="s")
    kfn = functools.partial(
        pl.kernel,
        out_type=(jax.ShapeDtypeStruct((NA, 128), jnp.float32),
                  jax.ShapeDtypeStruct((NB, 128), jnp.float32)),
        mesh=mesh,
        scratch_types=[
            pltpu.VMEM((G,), jnp.int32),
            pltpu.VMEM((G, 128), jnp.float32),
            pltpu.VMEM((G, 128), jnp.float32),
            pltpu.VMEM((G, 128), jnp.float32),
            pltpu.VMEM((G, 128), jnp.float32),
            pltpu.SemaphoreType.DMA,
        ],
    )(_gather_body)
    return kfn(qa, qb, e0_2d, e1_2d, e2_2d, fp_2d)


_FP_BLK = 2000


def _feat_proj_body(feat_ref, w_ref, out_ref):
    x = feat_ref[...]
    w = w_ref[...]
    proj = lax.dot_general(x, w, (((1,), (1,)), ((), ())),
                           preferred_element_type=jnp.float32)
    nrm = jnp.sqrt(jnp.sum(proj * proj, axis=1, keepdims=True))
    out_ref[...] = proj / (nrm + EPS)


def _feat_proj(feat_warm, W):
    grid = N_WARM // _FP_BLK
    return pl.pallas_call(
        _feat_proj_body,
        grid=(grid,),
        in_specs=[
            pl.BlockSpec((_FP_BLK, FEAT), lambda i: (i, 0)),
            pl.BlockSpec((EMB, FEAT), lambda i: (0, 0)),
        ],
        out_specs=pl.BlockSpec((_FP_BLK, EMB), lambda i: (i, 0)),
        out_shape=jax.ShapeDtypeStruct((N_WARM, EMB), jnp.float32),
    )(feat_warm, W)


_L_BLK = 512


def _log_sigmoid(x):
    return jnp.minimum(x, 0.0) - jnp.log1p(jnp.exp(-jnp.abs(x)))


def _sel(x128, hf):
    return jnp.where(hf < 0.5, x128[:, :EMB], x128[:, EMB:])


def _loss_body(u_ref, pos_ref, negt_ref, pf_ref, nft_ref,
               hu_ref, hp_ref, hn_ref, hnf_ref, out_ref):
    i = pl.program_id(0)
    u = _sel(u_ref[...], hu_ref[...])
    pos = _sel(pos_ref[...], hp_ref[...])
    pf = _sel(pf_ref[...], hp_ref[...])

    pos_scores = jnp.sum(u * pos, axis=1)
    unrm = jnp.sqrt(jnp.sum(u * u, axis=1, keepdims=True))
    u_n = u / (unrm + EPS)
    pos_sim = jnp.sum(u_n * pf, axis=1) / TAU

    bpr_sum = jnp.float32(0.0)
    reg_neg = jnp.float32(0.0)
    m = pos_sim
    nf_sims = []
    for j in range(N_NEGS):
        ne = _sel(negt_ref[j], hn_ref[j])
        ns = jnp.sum(u * ne, axis=1)
        bpr_sum += jnp.sum(_log_sigmoid(pos_scores - ns))
        reg_neg += jnp.sum(ne * ne)
        nf = _sel(nft_ref[j], hnf_ref[j])
        nfs = jnp.sum(u_n * nf, axis=1) / TAU
        nf_sims.append(nfs)
        m = jnp.maximum(m, nfs)
    s = jnp.exp(pos_sim - m)
    for j in range(N_NEGS):
        s += jnp.exp(nf_sims[j] - m)
    cl_sum = jnp.sum(jnp.log(s) + m - pos_sim)

    pnrm = jnp.sqrt(jnp.sum(pos * pos, axis=1, keepdims=True))
    pos_n = pos / (pnrm + EPS)
    diff = pos_n - pf
    align_sum = jnp.sum(diff * diff)

    reg_sum = jnp.sum(u * u) + jnp.sum(pos * pos) + reg_neg

    contrib = (-bpr_sum / (B * N_NEGS)
               + (LAMBDA_CL / B) * cl_sum
               + (ALIGN_W / (B * EMB)) * align_sum
               + (REG_W / B) * reg_sum)
    contrib2d = jnp.full((1, 1), 0.0, jnp.float32) + contrib

    @pl.when(i == 0)
    def _():
        out_ref[...] = contrib2d

    @pl.when(i > 0)
    def _():
        out_ref[...] = out_ref[...] + contrib2d


def _loss(u_p, pos_p, neg_p, pf_p, nf_p, hu, hp, hn, hnf):
    grid = B // _L_BLK
    out = pl.pallas_call(
        _loss_body,
        grid=(grid,),
        in_specs=[
            pl.BlockSpec((_L_BLK, 128), lambda i: (i, 0)),
            pl.BlockSpec((_L_BLK, 128), lambda i: (i, 0)),
            pl.BlockSpec((N_NEGS, _L_BLK, 128), lambda i: (0, i, 0)),
            pl.BlockSpec((_L_BLK, 128), lambda i: (i, 0)),
            pl.BlockSpec((N_NEGS, _L_BLK, 128), lambda i: (0, i, 0)),
            pl.BlockSpec((_L_BLK, 1), lambda i: (i, 0)),
            pl.BlockSpec((_L_BLK, 1), lambda i: (i, 0)),
            pl.BlockSpec((N_NEGS, _L_BLK, 1), lambda i: (0, i, 0)),
            pl.BlockSpec((N_NEGS, _L_BLK, 1), lambda i: (0, i, 0)),
        ],
        out_specs=pl.BlockSpec((1, 1), lambda i: (0, 0)),
        out_shape=jax.ShapeDtypeStruct((1, 1), jnp.float32),
    )(u_p, pos_p, neg_p, pf_p, nf_p, hu, hp, hn, hnf)
    return out[0, 0]


def kernel(users, pos_items, neg_items, feat_all, user_emb, item_emb, W,
           adj_row, adj_col, adj_val, neg_feat_idx):
    all_emb = jnp.concatenate([user_emb, item_emb], axis=0)
    e0_2d = all_emb.reshape(N_NODES * EMB // 128, 128)

    pad = PAD_LEN - NNZ
    row_p = jnp.concatenate(
        [adj_row.astype(jnp.int32), jnp.full((pad,), N_NODES, jnp.int32)])
    col_p = jnp.concatenate(
        [adj_col.astype(jnp.int32),
         (jnp.arange(pad, dtype=jnp.int32) % N_NODES)])
    val_p = jnp.concatenate([adj_val, jnp.zeros((pad,), jnp.float32)])

    e1_2d = _spmm(row_p, col_p, val_p, e0_2d).reshape(N_NODES * EMB // 128, 128)
    e2_2d = _spmm(row_p, col_p, val_p, e1_2d).reshape(N_NODES * EMB // 128, 128)

    users32 = users.astype(jnp.int32)
    pos32 = pos_items.astype(jnp.int32)
    neg32 = neg_items.astype(jnp.int32)
    negf32 = neg_feat_idx.astype(jnp.int32)

    idx_a = jnp.concatenate(
        [users32, pos32 + N_USERS, (neg32.T + N_USERS).reshape(-1)])
    idx_b = jnp.concatenate([pos32, negf32.T.reshape(-1)])
    qa, ha = idx_a >> 1, (idx_a & 1).astype(jnp.float32)
    qb, hb = idx_b >> 1, (idx_b & 1).astype(jnp.float32)

    fp = _feat_proj(feat_all[:N_WARM], W)
    fp_2d = fp.reshape(N_WARM * EMB // 128, 128)

    outa, outb = _gather_batch(qa, qb, e0_2d, e1_2d, e2_2d, fp_2d)

    u_p = outa[:B]
    pos_p = outa[B:2 * B]
    neg_p = outa[2 * B:].reshape(N_NEGS, B, 128)
    pf_p = outb[:B]
    nf_p = outb[B:].reshape(N_NEGS, B, 128)

    hu = ha[:B].reshape(B, 1)
    hp = ha[B:2 * B].reshape(B, 1)
    hn = ha[2 * B:].reshape(N_NEGS, B, 1)
    hnf = hb[B:].reshape(N_NEGS, B, 1)

    return _loss(u_p, pos_p, neg_p, pf_p, nf_p, hu, hp, hn, hnf)


# speedup vs baseline: 5.0134x; 1.0006x over previous
import functools

import jax
import jax.numpy as jnp
from jax import lax
from jax.experimental import pallas as pl
from jax.experimental.pallas import tpu as pltpu
from jax.experimental.pallas import tpu_sc as plsc

N_USERS = 50000
N_ITEMS = 50000
N_WARM = 40000
EMB = 64
FEAT = 256
N_LAYERS = 2
N_NEGS = 16
B = 4096
NNZ = 1200000
N_NODES = N_USERS + N_WARM
TAU = 0.2
LAMBDA_CL = 0.5
ALIGN_W = 0.1
REG_W = 1e-4
EPS = 1e-12

NW = 32
E_CH = 37504
K = 512
S = 896
PAD_LEN = NW * E_CH + K
ND = EMB // 16

NA = B * (2 + N_NEGS)
NB = B * (1 + N_NEGS)
CA = NA // NW
CB = NB // NW
G = 128


def _zero_window(staging):
    def zb(i, _):
        for u in range(4):
            staging[pl.ds(i * 64 + u * 16, 16)] = jnp.zeros((16,), jnp.float32)
        return 0
    lax.fori_loop(0, S * EMB // 64, zb, 0)


def _spmm_body(row_hbm, col_hbm, val_hbm, src_hbm, out_hbm,
               cols_v, colsq_v, rows_v, vals_v, gbuf, staging, scal16, sem):
    wid = lax.axis_index("s") * 2 + lax.axis_index("c")
    start = wid * E_CH

    off0 = pl.multiple_of(jnp.maximum(start - 16, 0), 16)
    pltpu.sync_copy(row_hbm.at[pl.ds(off0, 16)], scal16)
    prev = jnp.where(wid == 0, jnp.int32(-1), scal16[...][15])
    off1 = pl.multiple_of(start + E_CH - 16, 16)
    pltpu.sync_copy(row_hbm.at[pl.ds(off1, 16)], scal16)
    last = jnp.minimum(scal16[...][15], jnp.int32(N_NODES - 1))

    _zero_window(staging)

    def step(_, state):
        pos, wbase, done = state
        nd = done == 0
        wend = jnp.minimum(wbase + S, last + 1)
        pos_a = pl.multiple_of(pos, 128)

        @pl.when(nd)
        def _():
            cr = pltpu.async_copy(row_hbm.at[pl.ds(pos_a, K)], rows_v, sem)
            cv = pltpu.async_copy(val_hbm.at[pl.ds(pos_a, K)], vals_v, sem)
            cc = pltpu.async_copy(col_hbm.at[pl.ds(pos_a, K)], cols_v, sem)
            cr.wait(); cv.wait(); cc.wait()

            def shift_body(g, _):
                colsq_v[pl.ds(g * 16, 16)] = lax.shift_right_logical(
                    cols_v[pl.ds(g * 16, 16)], 1)
                return 0

            lax.fori_loop(0, K // 16, shift_body, 0)
            cps = [pltpu.async_copy(src_hbm.at[colsq_v.at[pl.ds(j * 128, 128)]],
                                    gbuf.at[pl.ds(j * 128, 128), :], sem)
                   for j in range(K // 128)]

            def group_body(g, carry):
                rvec = rows_v[pl.ds(g * 16, 16)]
                vvec = vals_v[pl.ds(g * 16, 16)]
                hvec = jnp.bitwise_and(cols_v[pl.ds(g * 16, 16)], jnp.int32(1))
                m = (rvec >= wbase) & (rvec < wend)
                vmask = jnp.where(m, vvec, jnp.float32(0.0))
                offv = jnp.clip(rvec - wbase, 0, S - 1) * EMB
                gbv = hvec * EMB
                for j in range(16):
                    vv = jnp.full((16,), vmask[j], jnp.float32)
                    e = g * 16 + j
                    base = pl.multiple_of(offv[j], 16)
                    gb = pl.multiple_of(gbv[j], 16)
                    for d in range(ND):
                        plsc.addupdate(staging.at[pl.ds(base + d * 16, 16)],
                                       vv * gbuf[e, pl.ds(gb + d * 16, 16)])
                return carry

            for h in range(K // 128):
                cps[h].wait()
                lax.fori_loop(h * 8, (h + 1) * 8, group_body, jnp.int32(0))

        batch_max = rows_v[pl.ds(K - 16, 16)][15]
        adv = batch_max >= wend

        @pl.when(nd & adv & (wbase + S <= last + 1))
        def _():
            dst = pl.multiple_of(wbase * EMB, 64)
            pltpu.async_copy(staging, out_hbm.at[pl.ds(dst, S * EMB)],
                             sem).wait()
            _zero_window(staging)

        @pl.when(nd & adv & (wbase + S > last + 1))
        def _():
            n = last + 1 - wbase
            offd = jnp.int32(0)
            for bit in reversed(range(10)):
                sz = 1 << bit
                take = (n & sz) != 0

                @pl.when(take)
                def _(offd=offd, sz=sz):
                    so = pl.multiple_of(offd * EMB, 64)
                    do = pl.multiple_of((wbase + offd) * EMB, 64)
                    pltpu.async_copy(staging.at[pl.ds(so, sz * EMB)],
                                     out_hbm.at[pl.ds(do, sz * EMB)],
                                     sem).wait()

                offd = offd + jnp.where(take, jnp.int32(sz), jnp.int32(0))

        new_pos = jnp.where(nd & jnp.logical_not(adv), pos_a + K, pos_a)
        new_wbase = jnp.where(nd & adv, wbase + S, wbase)
        new_done = jnp.where(new_wbase > last, jnp.int32(1), done)
        return new_pos, new_wbase, new_done

    T_STEPS = (E_CH + K - 1) // K + 1 + (N_NODES + S - 1) // S + 4
    lax.fori_loop(0, T_STEPS, step,
                  (start, prev + 1,
                   jnp.where(prev >= last, 1, 0).astype(jnp.int32)))


def _spmm(adj_row_pad, adj_col_pad, adj_val_pad, src_2d):
    mesh = plsc.VectorSubcoreMesh(core_axis_name="c", subcore_axis_name="s")
    kfn = functools.partial(
        pl.kernel,
        out_type=jax.ShapeDtypeStruct((N_NODES * EMB,), jnp.float32),
        mesh=mesh,
        scratch_types=[
            pltpu.VMEM((K,), jnp.int32),
            pltpu.VMEM((K,), jnp.int32),
            pltpu.VMEM((K,), jnp.int32),
            pltpu.VMEM((K,), jnp.float32),
            pltpu.VMEM((K, 128), jnp.float32),
            pltpu.VMEM((S * EMB,), jnp.float32),
            pltpu.VMEM((16,), jnp.int32),
            pltpu.SemaphoreType.DMA,
        ],
    )(_spmm_body)
    return kfn(adj_row_pad, adj_col_pad, adj_val_pad, src_2d)


def _gather_body(qa_hbm, qb_hbm, e0_hbm, e1_hbm, e2_hbm, fp_hbm,
                 outa_hbm, outb_hbm, idxb, g0, g1, g2, acc, sem):
    wid = lax.axis_index("s") * 2 + lax.axis_index("c")
    base_a = wid * CA
    base_b = wid * CB

    for t in range(CA // G):
        off = pl.multiple_of(base_a + t * G, 128)
        pltpu.sync_copy(qa_hbm.at[pl.ds(off, G)], idxb)
        c0 = pltpu.async_copy(e0_hbm.at[idxb], g0, sem)
        c1 = pltpu.async_copy(e1_hbm.at[idxb], g1, sem)
        c2 = pltpu.async_copy(e2_hbm.at[idxb], g2, sem)
        c0.wait(); c1.wait(); c2.wait()

        def avg_row(r, _):
            for u in range(8):
                sl = pl.ds(u * 16, 16)
                acc[r, sl] = (g0[r, sl] + g1[r, sl] + g2[r, sl]) * (1.0 / 3.0)
            return 0

        lax.fori_loop(0, G, avg_row, 0)
        pltpu.sync_copy(acc, outa_hbm.at[pl.ds(off, G), :])

    for t in range(CB // G):
        off = pl.multiple_of(base_b + t * G, 128)
        pltpu.sync_copy(qb_hbm.at[pl.ds(off, G)], idxb)
        pltpu.async_copy(fp_hbm.at[idxb], g0, sem).wait()
        pltpu.sync_copy(g0, outb_hbm.at[pl.ds(off, G), :])


def _gather_batch(qa, qb, e0_2d, e1_2d, e2_2d, fp_2d):
    mesh = plsc.VectorSubcoreMesh(core_axis_name="c", subcore_axis_name="s")
    kfn = functools.partial(
        pl.kernel,
        out_type=(jax.ShapeDtypeStruct((NA, 128), jnp.float32),
                  jax.ShapeDtypeStruct((NB, 128), jnp.float32)),
        mesh=mesh,
        scratch_types=[
            pltpu.VMEM((G,), jnp.int32),
            pltpu.VMEM((G, 128), jnp.float32),
            pltpu.VMEM((G, 128), jnp.float32),
            pltpu.VMEM((G, 128), jnp.float32),
            pltpu.VMEM((G, 128), jnp.float32),
            pltpu.SemaphoreType.DMA,
        ],
    )(_gather_body)
    return kfn(qa, qb, e0_2d, e1_2d, e2_2d, fp_2d)


_FP_BLK = 2000


def _feat_proj_body(feat_ref, w_ref, out_ref):
    x = feat_ref[...]
    w = w_ref[...]
    proj = lax.dot_general(x, w, (((1,), (1,)), ((), ())),
                           preferred_element_type=jnp.float32)
    nrm = jnp.sqrt(jnp.sum(proj * proj, axis=1, keepdims=True))
    out_ref[...] = proj / (nrm + EPS)


def _feat_proj(feat_warm, W):
    grid = N_WARM // _FP_BLK
    return pl.pallas_call(
        _feat_proj_body,
        grid=(grid,),
        in_specs=[
            pl.BlockSpec((_FP_BLK, FEAT), lambda i: (i, 0)),
            pl.BlockSpec((EMB, FEAT), lambda i: (0, 0)),
        ],
        out_specs=pl.BlockSpec((_FP_BLK, EMB), lambda i: (i, 0)),
        out_shape=jax.ShapeDtypeStruct((N_WARM, EMB), jnp.float32),
    )(feat_warm, W)


_L_BLK = 512


def _log_sigmoid(x):
    return jnp.minimum(x, 0.0) - jnp.log1p(jnp.exp(-jnp.abs(x)))


def _sel(x128, hf):
    return jnp.where(hf < 0.5, x128[:, :EMB], x128[:, EMB:])


def _loss_body(u_ref, pos_ref, negt_ref, pf_ref, nft_ref,
               hu_ref, hp_ref, hn_ref, hnf_ref, out_ref):
    i = pl.program_id(0)
    u = _sel(u_ref[...], hu_ref[...])
    pos = _sel(pos_ref[...], hp_ref[...])
    pf = _sel(pf_ref[...], hp_ref[...])

    pos_scores = jnp.sum(u * pos, axis=1)
    unrm = jnp.sqrt(jnp.sum(u * u, axis=1, keepdims=True))
    u_n = u / (unrm + EPS)
    pos_sim = jnp.sum(u_n * pf, axis=1) / TAU

    bpr_sum = jnp.float32(0.0)
    reg_neg = jnp.float32(0.0)
    m = pos_sim
    nf_sims = []
    for j in range(N_NEGS):
        ne = _sel(negt_ref[j], hn_ref[j])
        ns = jnp.sum(u * ne, axis=1)
        bpr_sum += jnp.sum(_log_sigmoid(pos_scores - ns))
        reg_neg += jnp.sum(ne * ne)
        nf = _sel(nft_ref[j], hnf_ref[j])
        nfs = jnp.sum(u_n * nf, axis=1) / TAU
        nf_sims.append(nfs)
        m = jnp.maximum(m, nfs)
    s = jnp.exp(pos_sim - m)
    for j in range(N_NEGS):
        s += jnp.exp(nf_sims[j] - m)
    cl_sum = jnp.sum(jnp.log(s) + m - pos_sim)

    pnrm = jnp.sqrt(jnp.sum(pos * pos, axis=1, keepdims=True))
    pos_n = pos / (pnrm + EPS)
    diff = pos_n - pf
    align_sum = jnp.sum(diff * diff)

    reg_sum = jnp.sum(u * u) + jnp.sum(pos * pos) + reg_neg

    contrib = (-bpr_sum / (B * N_NEGS)
               + (LAMBDA_CL / B) * cl_sum
               + (ALIGN_W / (B * EMB)) * align_sum
               + (REG_W / B) * reg_sum)
    contrib2d = jnp.full((1, 1), 0.0, jnp.float32) + contrib

    @pl.when(i == 0)
    def _():
        out_ref[...] = contrib2d

    @pl.when(i > 0)
    def _():
        out_ref[...] = out_ref[...] + contrib2d


def _loss(u_p, pos_p, neg_p, pf_p, nf_p, hu, hp, hn, hnf):
    grid = B // _L_BLK
    out = pl.pallas_call(
        _loss_body,
        grid=(grid,),
        in_specs=[
            pl.BlockSpec((_L_BLK, 128), lambda i: (i, 0)),
            pl.BlockSpec((_L_BLK, 128), lambda i: (i, 0)),
            pl.BlockSpec((N_NEGS, _L_BLK, 128), lambda i: (0, i, 0)),
            pl.BlockSpec((_L_BLK, 128), lambda i: (i, 0)),
            pl.BlockSpec((N_NEGS, _L_BLK, 128), lambda i: (0, i, 0)),
            pl.BlockSpec((_L_BLK, 1), lambda i: (i, 0)),
            pl.BlockSpec((_L_BLK, 1), lambda i: (i, 0)),
            pl.BlockSpec((N_NEGS, _L_BLK, 1), lambda i: (0, i, 0)),
            pl.BlockSpec((N_NEGS, _L_BLK, 1), lambda i: (0, i, 0)),
        ],
        out_specs=pl.BlockSpec((1, 1), lambda i: (0, 0)),
        out_shape=jax.ShapeDtypeStruct((1, 1), jnp.float32),
    )(u_p, pos_p, neg_p, pf_p, nf_p, hu, hp, hn, hnf)
    return out[0, 0]


def kernel(users, pos_items, neg_items, feat_all, user_emb, item_emb, W,
           adj_row, adj_col, adj_val, neg_feat_idx):
    all_emb = jnp.concatenate([user_emb, item_emb], axis=0)
    e0_2d = all_emb.reshape(N_NODES * EMB // 128, 128)

    pad = PAD_LEN - NNZ
    row_p = jnp.concatenate(
        [adj_row.astype(jnp.int32), jnp.full((pad,), N_NODES, jnp.int32)])
    col_p = jnp.concatenate(
        [adj_col.astype(jnp.int32),
         (jnp.arange(pad, dtype=jnp.int32) % N_NODES)])
    val_p = jnp.concatenate([adj_val, jnp.zeros((pad,), jnp.float32)])

    e1_2d = _spmm(row_p, col_p, val_p, e0_2d).reshape(N_NODES * EMB // 128, 128)
    e2_2d = _spmm(row_p, col_p, val_p, e1_2d).reshape(N_NODES * EMB // 128, 128)

    users32 = users.astype(jnp.int32)
    pos32 = pos_items.astype(jnp.int32)
    neg32 = neg_items.astype(jnp.int32)
    negf32 = neg_feat_idx.astype(jnp.int32)

    idx_a = jnp.concatenate(
        [users32, pos32 + N_USERS, (neg32.T + N_USERS).reshape(-1)])
    idx_b = jnp.concatenate([pos32, negf32.T.reshape(-1)])
    qa, ha = idx_a >> 1, (idx_a & 1).astype(jnp.float32)
    qb, hb = idx_b >> 1, (idx_b & 1).astype(jnp.float32)

    fp = _feat_proj(feat_all[:N_WARM], W)
    fp_2d = fp.reshape(N_WARM * EMB // 128, 128)

    outa, outb = _gather_batch(qa, qb, e0_2d, e1_2d, e2_2d, fp_2d)

    u_p = outa[:B]
    pos_p = outa[B:2 * B]
    neg_p = outa[2 * B:].reshape(N_NEGS, B, 128)
    pf_p = outb[:B]
    nf_p = outb[B:].reshape(N_NEGS, B, 128)

    hu = ha[:B].reshape(B, 1)
    hp = ha[B:2 * B].reshape(B, 1)
    hn = ha[2 * B:].reshape(N_NEGS, B, 1)
    hnf = hb[B:].reshape(N_NEGS, B, 1)

    return _loss(u_p, pos_p, neg_p, pf_p, nf_p, hu, hp, hn, hnf)
